# baseline (speedup 1.0000x reference)
import functools

import jax
import jax.numpy as jnp
from jax import lax
from jax.experimental import pallas as pl
from jax.experimental.pallas import tpu as pltpu
from jax.experimental.pallas import tpu_sc as plsc

_EMBED = 32
_NUM_WORKERS = 32
_ROW_BLOCK = 2048


def _sc_gather_body(s_hbm, d_hbm, table_hbm, s_out, d_out,
                    s_idx_v, d_idx_v, sem_s, sem_d):
  bpw = s_idx_v.shape[0]
  wid = lax.axis_index("s") * 2 + lax.axis_index("c")
  base = wid * bpw
  pltpu.sync_copy(s_hbm.at[pl.ds(base, bpw)], s_idx_v)
  pltpu.sync_copy(d_hbm.at[pl.ds(base, bpw)], d_idx_v)

  def issue(g, _):
    sv = s_idx_v[pl.ds(g * 16, 16)]
    dv = d_idx_v[pl.ds(g * 16, 16)]
    for lane in range(16):
      i = base + g * 16 + lane
      pltpu.make_async_copy(
          table_hbm.at[sv[lane]], s_out.at[i], sem_s).start()
      pltpu.make_async_copy(
          table_hbm.at[dv[lane]], d_out.at[i], sem_d).start()
    return ()

  lax.fori_loop(0, bpw // 16, issue, ())

  def drain(i, _):
    pltpu.make_async_copy(table_hbm.at[0], s_out.at[base + i], sem_s).wait()
    pltpu.make_async_copy(table_hbm.at[0], d_out.at[base + i], sem_d).wait()
    return ()

  lax.fori_loop(0, bpw, drain, ())


def _sc_gather(s_id, d_id, table):
  batch = s_id.shape[0]
  bpw = batch // _NUM_WORKERS
  mesh = plsc.VectorSubcoreMesh(core_axis_name="c", subcore_axis_name="s")
  out_ty = jax.ShapeDtypeStruct((batch, _EMBED), jnp.float32)
  fn = pl.kernel(
      _sc_gather_body,
      out_type=(out_ty, out_ty),
      mesh=mesh,
      scratch_types=[
          pltpu.VMEM((bpw,), jnp.int32),
          pltpu.VMEM((bpw,), jnp.int32),
          pltpu.SemaphoreType.DMA,
          pltpu.SemaphoreType.DMA,
      ],
  )
  return fn(s_id, d_id, table)


def _tc_compute_body(hs_ref, hd_ref, q_ref, out_ref):
  t = jnp.dot(hs_ref[...], q_ref[...], preferred_element_type=jnp.float32)
  out_ref[...] = jnp.exp(jnp.sum(t * hd_ref[...], axis=1))


def _tc_compute(h_s, h_d, Q):
  batch = h_s.shape[0]
  grid = batch // _ROW_BLOCK
  return pl.pallas_call(
      _tc_compute_body,
      grid=(grid,),
      in_specs=[
          pl.BlockSpec((_ROW_BLOCK, _EMBED), lambda i: (i, 0)),
          pl.BlockSpec((_ROW_BLOCK, _EMBED), lambda i: (i, 0)),
          pl.BlockSpec((_EMBED, _EMBED), lambda i: (0, 0)),
      ],
      out_specs=pl.BlockSpec((_ROW_BLOCK,), lambda i: (i,)),
      out_shape=jax.ShapeDtypeStruct((batch,), jnp.float32),
  )(h_s, h_d, Q)


@jax.jit
def kernel(s_id, d_id, h_static, Q):
  h_s, h_d = _sc_gather(s_id.astype(jnp.int32), d_id.astype(jnp.int32),
                        h_static)
  return _tc_compute(h_s, h_d, Q)


# speedup vs baseline: 2.0755x; 2.0755x over previous
import functools

import jax
import jax.numpy as jnp
from jax import lax
from jax.experimental import pallas as pl
from jax.experimental.pallas import tpu as pltpu
from jax.experimental.pallas import tpu_sc as plsc

_N = 1000001
_E = 32
_B = 16384
_W = 32
_CH = 1024
_NCH = 976
_TAIL = 999424
_CAP = 784
_SLOT = 1536
_CCAP = 272
_PIECE = 2048
_DEAD = 1 << 20
_BLK = 2048


def _iota16():
  return lax.iota(jnp.int32, 16)


def _sc_gather_body(s_hbm, d_hbm, tab_hbm,
                    rows_s_out, rows_d_out, bpos_s_out, bpos_d_out,
                    chunk0, chunk1, piece,
                    wl_idx_s, wl_bpos_s, wl_idx_d, wl_bpos_d,
                    cwl_loc, cwl_bpos, grp,
                    sem_p, sem_c0, sem_c1, sem_f):
  w = lax.axis_index("s") * 2 + lax.axis_index("c")
  it16 = _iota16()

  sent_bp = _B + it16
  for v in range(_CAP // 16):
    wl_bpos_s[pl.ds(v * 16, 16)] = sent_bp
    wl_idx_s[pl.ds(v * 16, 16)] = jnp.full((16,), _DEAD, jnp.int32)
    wl_idx_d[pl.ds(v * 16, 16)] = jnp.full((16,), _DEAD, jnp.int32)
  for v in range(_CCAP // 16):
    cwl_loc[pl.ds(v * 16, 16)] = jnp.zeros((16,), jnp.int32)
  for h in range(2):
    pltpu.sync_copy(wl_bpos_s.at[pl.ds(0, _SLOT // 2)],
                    bpos_s_out.at[pl.ds(w * _SLOT + h * (_SLOT // 2),
                                        _SLOT // 2)])
    pltpu.sync_copy(wl_bpos_s.at[pl.ds(0, _SLOT // 2)],
                    bpos_d_out.at[pl.ds(w * _SLOT + h * (_SLOT // 2),
                                        _SLOT // 2)])
  for v in range(_CAP // 16):
    wl_bpos_d[pl.ds(v * 16, 16)] = sent_bp

  def build(idx_hbm, wl_idx, wl_bpos):
    n = jnp.int32(0)
    for p in range(_B // _PIECE):
      pltpu.sync_copy(idx_hbm.at[pl.ds(p * _PIECE, _PIECE)], piece)

      def scan_vec(v, off):
        idx = piece[pl.ds(v * 16, 16)]
        m = ((idx >> 10) & 31) == w
        pos = off + plsc.cumsum(m.astype(jnp.int32)) - 1
        plsc.store_scatter(wl_idx, [pos], idx, mask=m)
        bp = jnp.full((16,), p * _PIECE, jnp.int32) + v * 16 + it16
        plsc.store_scatter(wl_bpos, [pos], bp, mask=m)
        return off + plsc.all_reduce_population_count(m)[0]

      n = lax.fori_loop(0, _PIECE // 16, scan_vec, n)
    return n

  n_s = build(s_hbm, wl_idx_s, wl_bpos_s)
  n_d = build(d_hbm, wl_idx_d, wl_bpos_d)

  def fire(k, buf, sem):
    c = w + 32 * k
    pltpu.make_async_copy(
        tab_hbm.at[:, pl.ds(c * _CH, _CH)], buf, sem).start()

  def process(c, buf, wl_idx, wl_bpos, n_wl, tot, rows_out, bpos_out):
    for v in range(_CCAP // 16):
      cwl_bpos[pl.ds(v * 16, 16)] = sent_bp

    def scan_wl(v, off2):
      wi = wl_idx[pl.ds(v * 16, 16)]
      m = (wi >> 10) == c
      pos = off2 + plsc.cumsum(m.astype(jnp.int32)) - 1
      plsc.store_scatter(cwl_loc, [pos], wi & 1023, mask=m)
      bp = wl_bpos[pl.ds(v * 16, 16)]
      plsc.store_scatter(cwl_bpos, [pos], bp, mask=m)
      return off2 + plsc.all_reduce_population_count(m)[0]

    n_vecs = (n_wl + 15) >> 4
    off2 = lax.fori_loop(0, n_vecs, scan_wl, jnp.int32(0))

    def do_group(g, _):
      base = pl.multiple_of(w * _SLOT + tot + g * 16, 16)
      loc16 = cwl_loc[pl.ds(g * 16, 16)]
      for j in range(_E):
        vals = plsc.load_gather(
            buf, [jnp.full((16,), j, jnp.int32), loc16])
        plsc.store_scatter(grp, [it16 * _E + j], vals)
      dst = pl.multiple_of(base * _E, 16 * _E)
      pltpu.make_async_copy(
          grp, rows_out.at[pl.ds(dst, 16 * _E)], sem_f).start()
      pltpu.make_async_copy(
          cwl_bpos.at[pl.ds(g * 16, 16)],
          bpos_out.at[pl.ds(base, 16)], sem_f).start()
      pltpu.make_async_copy(
          grp, rows_out.at[pl.ds(dst, 16 * _E)], sem_f).wait()
      pltpu.make_async_copy(
          cwl_bpos.at[pl.ds(g * 16, 16)],
          bpos_out.at[pl.ds(base, 16)], sem_f).wait()
      return ()

    lax.fori_loop(0, (off2 + 15) >> 4, do_group, ())
    return tot + ((off2 + 15) & ~15)

  bufs = (chunk0, chunk1)
  sems = (sem_c0, sem_c1)
  fire(0, bufs[0], sems[0])
  fire(1, bufs[1], sems[1])

  def chunk_pair(kk, carry):
    tot_s, tot_d = carry
    for par in range(2):
      k = 2 * kk + par
      b, sm = bufs[par], sems[par]
      pltpu.make_async_copy(tab_hbm.at[:, pl.ds(0, _CH)], b, sm).wait()
      c = w + 32 * k
      tot_s = process(c, b, wl_idx_s, wl_bpos_s, n_s, tot_s,
                      rows_s_out, bpos_s_out)
      tot_d = process(c, b, wl_idx_d, wl_bpos_d, n_d, tot_d,
                      rows_d_out, bpos_d_out)

      @pl.when(w + 32 * (k + 2) < _NCH)
      def _():
        fire(k + 2, b, sm)
    return tot_s, tot_d

  tot_s, tot_d = lax.fori_loop(0, 15, chunk_pair,
                               (jnp.int32(0), jnp.int32(0)))

  @pl.when(w < 16)
  def _():
    b, sm = bufs[0], sems[0]
    pltpu.make_async_copy(tab_hbm.at[:, pl.ds(0, _CH)], b, sm).wait()
    c = w + 32 * 30
    ts = process(c, b, wl_idx_s, wl_bpos_s, n_s, tot_s,
                 rows_s_out, bpos_s_out)
    td = process(c, b, wl_idx_d, wl_bpos_d, n_d, tot_d,
                 rows_d_out, bpos_d_out)
    del ts, td


def _sc_gather(s_id, d_id, tab_t):
  mesh = plsc.VectorSubcoreMesh(core_axis_name="c", subcore_axis_name="s")
  rows_ty = jax.ShapeDtypeStruct((_W * _SLOT * _E,), jnp.float32)
  bpos_ty = jax.ShapeDtypeStruct((_W * _SLOT,), jnp.int32)
  fn = pl.kernel(
      _sc_gather_body,
      out_type=(rows_ty, rows_ty, bpos_ty, bpos_ty),
      mesh=mesh,
      compiler_params=pltpu.CompilerParams(needs_layout_passes=False),
      scratch_types=[
          pltpu.VMEM((_E, _CH), jnp.float32),
          pltpu.VMEM((_E, _CH), jnp.float32),
          pltpu.VMEM((_PIECE,), jnp.int32),
          pltpu.VMEM((_CAP,), jnp.int32),
          pltpu.VMEM((_CAP,), jnp.int32),
          pltpu.VMEM((_CAP,), jnp.int32),
          pltpu.VMEM((_CAP,), jnp.int32),
          pltpu.VMEM((_CCAP,), jnp.int32),
          pltpu.VMEM((_CCAP,), jnp.int32),
          pltpu.VMEM((16 * _E,), jnp.float32),
          pltpu.SemaphoreType.DMA,
          pltpu.SemaphoreType.DMA,
          pltpu.SemaphoreType.DMA,
          pltpu.SemaphoreType.DMA,
      ],
  )
  return fn(s_id, d_id, tab_t)


def _sc_scatter_body(rows_s, rows_d, bpos_s, bpos_d, hs_out, hd_out,
                     rows_v, bpv, sem):
  w = lax.axis_index("s") * 2 + lax.axis_index("c")
  for st, (rows, bpos, out) in enumerate(
      ((rows_s, bpos_s, hs_out), (rows_d, bpos_d, hd_out))):
    for j in range(_SLOT // 128):
      pltpu.sync_copy(rows.at[pl.ds(w * _SLOT + j * 128, 128)],
                      rows_v.at[j])
      pltpu.sync_copy(bpos.at[pl.ds(w * _SLOT + j * 128, 128)], bpv.at[j])
    for j in range(_SLOT // 128):
      pltpu.make_async_copy(rows_v.at[j], out.at[bpv.at[j]], sem).start()
    for j in range(_SLOT // 128):
      pltpu.make_async_copy(rows_v.at[j], out.at[bpv.at[j]], sem).wait()


def _sc_scatter(rows_s, rows_d, bpos_s, bpos_d):
  mesh = plsc.VectorSubcoreMesh(core_axis_name="c", subcore_axis_name="s")
  out_ty = jax.ShapeDtypeStruct((_B + 16, _E), jnp.float32)
  fn = pl.kernel(
      _sc_scatter_body,
      out_type=(out_ty, out_ty),
      mesh=mesh,
      compiler_params=pltpu.CompilerParams(use_tc_tiling_on_sc=False,
                                           needs_layout_passes=False),
      scratch_types=[
          pltpu.VMEM((_SLOT // 128, 128, _E), jnp.float32),
          pltpu.VMEM((_SLOT // 128, 128), jnp.int32),
          pltpu.SemaphoreType.DMA,
      ],
  )
  return fn(rows_s.reshape(_W * _SLOT, _E), rows_d.reshape(_W * _SLOT, _E),
            bpos_s, bpos_d)


def _tc_compute_body(hs_ref, hd_ref, q_ref, sid_ref, did_ref, tail_ref,
                     out_ref):
  it = lax.broadcasted_iota(jnp.int32, (_BLK, 640), 1)

  def patch(h, ids):
    os = ids[...] - _TAIL
    oh = (os == it).astype(jnp.float32)
    fix = jnp.dot(oh, tail_ref[...], preferred_element_type=jnp.float32)
    return jnp.where(os >= 0, fix, h[...])

  hs = patch(hs_ref, sid_ref)
  hd = patch(hd_ref, did_ref)
  t = jnp.dot(hs, q_ref[...], preferred_element_type=jnp.float32)
  out_ref[...] = jnp.exp(jnp.sum(t * hd, axis=1))


def _tc_compute(h_s, h_d, Q, s_id, d_id, h_tail):
  grid = _B // _BLK
  return pl.pallas_call(
      _tc_compute_body,
      grid=(grid,),
      in_specs=[
          pl.BlockSpec((_BLK, _E), lambda i: (i, 0)),
          pl.BlockSpec((_BLK, _E), lambda i: (i, 0)),
          pl.BlockSpec((_E, _E), lambda i: (0, 0)),
          pl.BlockSpec((_BLK, 1), lambda i: (i, 0)),
          pl.BlockSpec((_BLK, 1), lambda i: (i, 0)),
          pl.BlockSpec((640, _E), lambda i: (0, 0)),
      ],
      out_specs=pl.BlockSpec((_BLK,), lambda i: (i,)),
      out_shape=jax.ShapeDtypeStruct((_B,), jnp.float32),
  )(h_s, h_d, Q, s_id.reshape(_B, 1), d_id.reshape(_B, 1), h_tail)


@jax.jit
def kernel(s_id, d_id, h_static, Q):
  s_id = s_id.astype(jnp.int32)
  d_id = d_id.astype(jnp.int32)
  tab_t = h_static.T
  rows_s, rows_d, bpos_s, bpos_d = _sc_gather(s_id, d_id, tab_t)
  hs_pad, hd_pad = _sc_scatter(rows_s, rows_d, bpos_s, bpos_d)
  h_tail = jnp.zeros((640, _E), jnp.float32).at[: _N - _TAIL].set(
      h_static[_TAIL:])
  return _tc_compute(hs_pad[:_B], hd_pad[:_B], Q, s_id, d_id, h_tail)


# speedup vs baseline: 3.0186x; 1.4544x over previous
import functools

import jax
import jax.numpy as jnp
from jax import lax
from jax.experimental import pallas as pl
from jax.experimental.pallas import tpu as pltpu
from jax.experimental.pallas import tpu_sc as plsc

_N = 1000001
_E = 32
_B = 16384
_W = 32
_CH = 1024
_NCH = 976
_TAIL = 999424
_CAP = 784
_SLOT = 1536
_CCAP = 272
_PIECE = 2048
_DEAD = 1 << 20
_BLK = 2048


def _iota16():
  return lax.iota(jnp.int32, 16)


def _sc_gather_body(s_hbm, d_hbm, tab_hbm,
                    rows_s_out, rows_d_out, bpos_s_out, bpos_d_out,
                    wts_out, wtd_out,
                    chunk0, chunk1, piece,
                    wl_idx_s, wl_bpos_s, wl_idx_d, wl_bpos_d,
                    cwl_loc, cwl_bpos, grp,
                    sem_p, sem_c0, sem_c1, sem_f):
  w = lax.axis_index("s") * 2 + lax.axis_index("c")
  it16 = _iota16()

  sent_bp = _B + it16
  for v in range(_CAP // 16):
    wl_bpos_s[pl.ds(v * 16, 16)] = sent_bp
    wl_idx_s[pl.ds(v * 16, 16)] = jnp.full((16,), _DEAD, jnp.int32)
    wl_idx_d[pl.ds(v * 16, 16)] = jnp.full((16,), _DEAD, jnp.int32)
  for v in range(_CCAP // 16):
    cwl_loc[pl.ds(v * 16, 16)] = jnp.zeros((16,), jnp.int32)
  for h in range(2):
    pltpu.sync_copy(wl_bpos_s.at[pl.ds(0, _SLOT // 2)],
                    bpos_s_out.at[pl.ds(w * _SLOT + h * (_SLOT // 2),
                                        _SLOT // 2)])
    pltpu.sync_copy(wl_bpos_s.at[pl.ds(0, _SLOT // 2)],
                    bpos_d_out.at[pl.ds(w * _SLOT + h * (_SLOT // 2),
                                        _SLOT // 2)])
  for v in range(_CAP // 16):
    wl_bpos_d[pl.ds(v * 16, 16)] = sent_bp

  def build(idx_hbm, wl_idx, wl_bpos):
    n = jnp.int32(0)
    for p in range(_B // _PIECE):
      pltpu.sync_copy(idx_hbm.at[pl.ds(p * _PIECE, _PIECE)], piece)

      def scan_vec(v, off):
        idx = piece[pl.ds(v * 16, 16)]
        m = ((idx >> 10) & 31) == w
        pos = off + plsc.cumsum(m.astype(jnp.int32)) - 1
        plsc.store_scatter(wl_idx, [pos], idx, mask=m)
        bp = jnp.full((16,), p * _PIECE, jnp.int32) + v * 16 + it16
        plsc.store_scatter(wl_bpos, [pos], bp, mask=m)
        return off + plsc.all_reduce_population_count(m)[0]

      n = lax.fori_loop(0, _PIECE // 16, scan_vec, n)
    return n

  n_s = build(s_hbm, wl_idx_s, wl_bpos_s)
  n_d = build(d_hbm, wl_idx_d, wl_bpos_d)

  def fire(k, buf, sem):
    c = w + 32 * k
    pltpu.make_async_copy(
        tab_hbm.at[:, pl.ds(c * _CH, _CH)], buf, sem).start()

  def process(c, buf, wl_idx, wl_bpos, n_wl, tot, rows_out, bpos_out):
    for v in range(_CCAP // 16):
      cwl_bpos[pl.ds(v * 16, 16)] = sent_bp

    def scan_wl(v, off2):
      wi = wl_idx[pl.ds(v * 16, 16)]
      m = (wi >> 10) == c
      pos = off2 + plsc.cumsum(m.astype(jnp.int32)) - 1
      plsc.store_scatter(cwl_loc, [pos], wi & 1023, mask=m)
      bp = wl_bpos[pl.ds(v * 16, 16)]
      plsc.store_scatter(cwl_bpos, [pos], bp, mask=m)
      return off2 + plsc.all_reduce_population_count(m)[0]

    n_vecs = (n_wl + 15) >> 4
    off2 = lax.fori_loop(0, n_vecs, scan_wl, jnp.int32(0))

    def do_group(g, _):
      base = pl.multiple_of(w * _SLOT + tot + g * 16, 16)
      loc16 = cwl_loc[pl.ds(g * 16, 16)]
      for j in range(_E):
        vals = plsc.load_gather(
            buf, [jnp.full((16,), j, jnp.int32), loc16])
        plsc.store_scatter(grp, [it16 * _E + j], vals)
      dst = pl.multiple_of(base * _E, 16 * _E)
      pltpu.make_async_copy(
          grp, rows_out.at[pl.ds(dst, 16 * _E)], sem_f).start()
      pltpu.make_async_copy(
          cwl_bpos.at[pl.ds(g * 16, 16)],
          bpos_out.at[pl.ds(base, 16)], sem_f).start()
      pltpu.make_async_copy(
          grp, rows_out.at[pl.ds(dst, 16 * _E)], sem_f).wait()
      pltpu.make_async_copy(
          cwl_bpos.at[pl.ds(g * 16, 16)],
          bpos_out.at[pl.ds(base, 16)], sem_f).wait()
      return ()

    lax.fori_loop(0, (off2 + 15) >> 4, do_group, ())
    return tot + ((off2 + 15) & ~15)

  bufs = (chunk0, chunk1)
  sems = (sem_c0, sem_c1)
  fire(0, bufs[0], sems[0])
  fire(1, bufs[1], sems[1])

  def chunk_pair(kk, carry):
    tot_s, tot_d = carry
    for par in range(2):
      k = 2 * kk + par
      b, sm = bufs[par], sems[par]
      pltpu.make_async_copy(tab_hbm.at[:, pl.ds(0, _CH)], b, sm).wait()
      c = w + 32 * k
      tot_s = process(c, b, wl_idx_s, wl_bpos_s, n_s, tot_s,
                      rows_s_out, bpos_s_out)
      tot_d = process(c, b, wl_idx_d, wl_bpos_d, n_d, tot_d,
                      rows_d_out, bpos_d_out)

      @pl.when(w + 32 * (k + 2) < _NCH)
      def _():
        fire(k + 2, b, sm)
    return tot_s, tot_d

  tot_s, tot_d = lax.fori_loop(0, 15, chunk_pair,
                               (jnp.int32(0), jnp.int32(0)))

  def write_tots(ts, td):
    grp[pl.ds(0, 16)] = jnp.full((16,), ts, jnp.int32).astype(jnp.float32)
    grp[pl.ds(16, 16)] = jnp.full((16,), td, jnp.int32).astype(jnp.float32)
    pltpu.sync_copy(grp.at[pl.ds(0, 8)], wts_out.at[pl.ds(w * 8, 8)])
    pltpu.sync_copy(grp.at[pl.ds(16, 8)], wtd_out.at[pl.ds(w * 8, 8)])

  @pl.when(w < 16)
  def _():
    b, sm = bufs[0], sems[0]
    pltpu.make_async_copy(tab_hbm.at[:, pl.ds(0, _CH)], b, sm).wait()
    c = w + 32 * 30
    ts = process(c, b, wl_idx_s, wl_bpos_s, n_s, tot_s,
                 rows_s_out, bpos_s_out)
    td = process(c, b, wl_idx_d, wl_bpos_d, n_d, tot_d,
                 rows_d_out, bpos_d_out)
    write_tots(ts, td)

  @pl.when(w >= 16)
  def _():
    write_tots(tot_s, tot_d)


def _sc_gather(s_id, d_id, tab_t):
  mesh = plsc.VectorSubcoreMesh(core_axis_name="c", subcore_axis_name="s")
  rows_ty = jax.ShapeDtypeStruct((_W * _SLOT * _E,), jnp.float32)
  bpos_ty = jax.ShapeDtypeStruct((_W * _SLOT,), jnp.int32)
  cnt_ty = jax.ShapeDtypeStruct((_W * 8,), jnp.float32)
  fn = pl.kernel(
      _sc_gather_body,
      out_type=(rows_ty, rows_ty, bpos_ty, bpos_ty, cnt_ty, cnt_ty),
      mesh=mesh,
      compiler_params=pltpu.CompilerParams(needs_layout_passes=False),
      scratch_types=[
          pltpu.VMEM((_E, _CH), jnp.float32),
          pltpu.VMEM((_E, _CH), jnp.float32),
          pltpu.VMEM((_PIECE,), jnp.int32),
          pltpu.VMEM((_CAP,), jnp.int32),
          pltpu.VMEM((_CAP,), jnp.int32),
          pltpu.VMEM((_CAP,), jnp.int32),
          pltpu.VMEM((_CAP,), jnp.int32),
          pltpu.VMEM((_CCAP,), jnp.int32),
          pltpu.VMEM((_CCAP,), jnp.int32),
          pltpu.VMEM((16 * _E,), jnp.float32),
          pltpu.SemaphoreType.DMA,
          pltpu.SemaphoreType.DMA,
          pltpu.SemaphoreType.DMA,
          pltpu.SemaphoreType.DMA,
      ],
  )
  return fn(s_id, d_id, tab_t)


def _sc_scatter_body(rows_s, rows_d, bpos_s, bpos_d, wts, wtd,
                     hs_out, hd_out, rows_v, bpv, cnt_v, sem, sem2):
  w = lax.axis_index("s") * 2 + lax.axis_index("c")
  pltpu.sync_copy(wts.at[pl.ds(w * 8, 8)], cnt_v.at[pl.ds(0, 8)])
  pltpu.sync_copy(wtd.at[pl.ds(w * 8, 8)], cnt_v.at[pl.ds(8, 8)])
  cnts = cnt_v[pl.ds(0, 16)].astype(jnp.int32)
  for st, (rows, bpos, out, hs_half) in enumerate(
      ((rows_s, bpos_s, hs_out, 0), (rows_d, bpos_d, hd_out, 1))):
    ng = (cnts[8 * st] + 127) >> 7

    def stage_and_fire(j, _):
      pltpu.make_async_copy(
          rows.at[pl.ds(w * _SLOT + j * 128, 128)],
          rows_v.at[st * (_SLOT // 128) + j], sem2).start()
      pltpu.make_async_copy(
          bpos.at[pl.ds(w * _SLOT + j * 128, 128)],
          bpv.at[st * (_SLOT // 128) + j], sem2).start()
      pltpu.make_async_copy(
          rows.at[pl.ds(w * _SLOT + j * 128, 128)],
          rows_v.at[st * (_SLOT // 128) + j], sem2).wait()
      pltpu.make_async_copy(
          bpos.at[pl.ds(w * _SLOT + j * 128, 128)],
          bpv.at[st * (_SLOT // 128) + j], sem2).wait()
      pltpu.make_async_copy(
          rows_v.at[st * (_SLOT // 128) + j],
          out.at[bpv.at[st * (_SLOT // 128) + j]], sem).start()
      return ()

    lax.fori_loop(0, ng, stage_and_fire, ())

    def drain(j, _):
      pltpu.make_async_copy(
          rows_v.at[st * (_SLOT // 128) + j],
          out.at[bpv.at[st * (_SLOT // 128) + j]], sem).wait()
      return ()

    lax.fori_loop(0, ng, drain, ())


def _sc_scatter(rows_s, rows_d, bpos_s, bpos_d, wts, wtd):
  mesh = plsc.VectorSubcoreMesh(core_axis_name="c", subcore_axis_name="s")
  out_ty = jax.ShapeDtypeStruct((_B + 16, _E), jnp.float32)
  fn = pl.kernel(
      _sc_scatter_body,
      out_type=(out_ty, out_ty),
      mesh=mesh,
      compiler_params=pltpu.CompilerParams(use_tc_tiling_on_sc=False,
                                           needs_layout_passes=False),
      scratch_types=[
          pltpu.VMEM((2 * (_SLOT // 128), 128, _E), jnp.float32),
          pltpu.VMEM((2 * (_SLOT // 128), 128), jnp.int32),
          pltpu.VMEM((16,), jnp.float32),
          pltpu.SemaphoreType.DMA,
          pltpu.SemaphoreType.DMA,
      ],
  )
  return fn(rows_s.reshape(_W * _SLOT, _E), rows_d.reshape(_W * _SLOT, _E),
            bpos_s, bpos_d, wts, wtd)


def _tc_compute_body(hs_ref, hd_ref, q_ref, sid_ref, did_ref, tail_ref,
                     out_ref):
  it = lax.broadcasted_iota(jnp.int32, (_BLK, 640), 1)

  def patch(h, ids):
    os = ids[...] - _TAIL
    oh = (os == it).astype(jnp.float32)
    fix = jnp.dot(oh, tail_ref[...], preferred_element_type=jnp.float32)
    return jnp.where(os >= 0, fix, h[...])

  hs = patch(hs_ref, sid_ref)
  hd = patch(hd_ref, did_ref)
  t = jnp.dot(hs, q_ref[...], preferred_element_type=jnp.float32)
  out_ref[...] = jnp.exp(jnp.sum(t * hd, axis=1))


def _tc_compute(h_s, h_d, Q, s_id, d_id, h_tail):
  grid = _B // _BLK
  return pl.pallas_call(
      _tc_compute_body,
      grid=(grid,),
      in_specs=[
          pl.BlockSpec((_BLK, _E), lambda i: (i, 0)),
          pl.BlockSpec((_BLK, _E), lambda i: (i, 0)),
          pl.BlockSpec((_E, _E), lambda i: (0, 0)),
          pl.BlockSpec((_BLK, 1), lambda i: (i, 0)),
          pl.BlockSpec((_BLK, 1), lambda i: (i, 0)),
          pl.BlockSpec((640, _E), lambda i: (0, 0)),
      ],
      out_specs=pl.BlockSpec((_BLK,), lambda i: (i,)),
      out_shape=jax.ShapeDtypeStruct((_B,), jnp.float32),
  )(h_s, h_d, Q, s_id.reshape(_B, 1), d_id.reshape(_B, 1), h_tail)


@jax.jit
def kernel(s_id, d_id, h_static, Q):
  s_id = s_id.astype(jnp.int32)
  d_id = d_id.astype(jnp.int32)
  tab_t = h_static.T
  rows_s, rows_d, bpos_s, bpos_d, wts, wtd = _sc_gather(s_id, d_id, tab_t)
  hs_pad, hd_pad = _sc_scatter(rows_s, rows_d, bpos_s, bpos_d, wts, wtd)
  h_tail = jnp.zeros((640, _E), jnp.float32).at[: _N - _TAIL].set(
      h_static[_TAIL:])
  return _tc_compute(hs_pad[:_B], hd_pad[:_B], Q, s_id, d_id, h_tail)


# speedup vs baseline: 3.0422x; 1.0078x over previous
import functools

import jax
import jax.numpy as jnp
from jax import lax
from jax.experimental import pallas as pl
from jax.experimental.pallas import tpu as pltpu
from jax.experimental.pallas import tpu_sc as plsc

_N = 1000001
_E = 32
_B = 16384
_W = 32
_CH = 1024
_NCH = 976
_TAIL = 999424
_CAP = 784
_SLOT = 1536
_CCAP = 272
_PIECE = 2048
_DEAD = 1 << 20
_BLK = 2048


def _iota16():
  return lax.iota(jnp.int32, 16)


def _sc_gather_body(s_hbm, d_hbm, tab_hbm,
                    rows_s_out, rows_d_out, bpos_s_out, bpos_d_out,
                    wts_out, wtd_out,
                    chunk0, chunk1, piece,
                    wl_idx_s, wl_bpos_s, wl_idx_d, wl_bpos_d,
                    cwl_loc, cwl_bpos, grp,
                    sem_p, sem_c0, sem_c1, sem_f):
  w = lax.axis_index("s") * 2 + lax.axis_index("c")
  it16 = _iota16()

  sent_bp = _B + it16
  for v in range(_CAP // 16):
    wl_bpos_s[pl.ds(v * 16, 16)] = sent_bp
    wl_idx_s[pl.ds(v * 16, 16)] = jnp.full((16,), _DEAD, jnp.int32)
    wl_idx_d[pl.ds(v * 16, 16)] = jnp.full((16,), _DEAD, jnp.int32)
  for v in range(_CCAP // 16):
    cwl_loc[pl.ds(v * 16, 16)] = jnp.zeros((16,), jnp.int32)
  for h in range(2):
    pltpu.sync_copy(wl_bpos_s.at[pl.ds(0, _SLOT // 2)],
                    bpos_s_out.at[pl.ds(w * _SLOT + h * (_SLOT // 2),
                                        _SLOT // 2)])
    pltpu.sync_copy(wl_bpos_s.at[pl.ds(0, _SLOT // 2)],
                    bpos_d_out.at[pl.ds(w * _SLOT + h * (_SLOT // 2),
                                        _SLOT // 2)])
  for v in range(_CAP // 16):
    wl_bpos_d[pl.ds(v * 16, 16)] = sent_bp

  def build(idx_hbm, wl_idx, wl_bpos):
    n = jnp.int32(0)
    for p in range(_B // _PIECE):
      pltpu.sync_copy(idx_hbm.at[pl.ds(p * _PIECE, _PIECE)], piece)

      def scan_vec(v, off):
        idx = piece[pl.ds(v * 16, 16)]
        m = ((idx >> 10) & 31) == w
        pos = off + plsc.cumsum(m.astype(jnp.int32)) - 1
        plsc.store_scatter(wl_idx, [pos], idx, mask=m)
        bp = jnp.full((16,), p * _PIECE, jnp.int32) + v * 16 + it16
        plsc.store_scatter(wl_bpos, [pos], bp, mask=m)
        return off + plsc.all_reduce_population_count(m)[0]

      n = lax.fori_loop(0, _PIECE // 16, scan_vec, n)
    return n

  n_s = build(s_hbm, wl_idx_s, wl_bpos_s)
  n_d = build(d_hbm, wl_idx_d, wl_bpos_d)

  def fire(k, buf, sem):
    c = w + 32 * k
    pltpu.make_async_copy(
        tab_hbm.at[:, pl.ds(c * _CH, _CH)], buf, sem).start()

  def process(c, buf, wl_idx, wl_bpos, n_wl, tot, rows_out, bpos_out):
    for v in range(_CCAP // 16):
      cwl_bpos[pl.ds(v * 16, 16)] = sent_bp

    def scan_wl(v, off2):
      wi = wl_idx[pl.ds(v * 16, 16)]
      m = (wi >> 10) == c
      pos = off2 + plsc.cumsum(m.astype(jnp.int32)) - 1
      plsc.store_scatter(cwl_loc, [pos], wi & 1023, mask=m)
      bp = wl_bpos[pl.ds(v * 16, 16)]
      plsc.store_scatter(cwl_bpos, [pos], bp, mask=m)
      return off2 + plsc.all_reduce_population_count(m)[0]

    n_vecs = (n_wl + 15) >> 4
    off2 = lax.fori_loop(0, n_vecs, scan_wl, jnp.int32(0))

    def do_group(g, _):
      base = pl.multiple_of(w * _SLOT + tot + g * 16, 16)
      loc16 = cwl_loc[pl.ds(g * 16, 16)]
      for j in range(_E):
        vals = plsc.load_gather(
            buf, [jnp.full((16,), j, jnp.int32), loc16])
        plsc.store_scatter(grp, [g * (16 * _E) + it16 * _E + j], vals)
      dst = pl.multiple_of(base * _E, 16 * _E)
      pltpu.make_async_copy(
          grp.at[pl.ds(g * (16 * _E), 16 * _E)],
          rows_out.at[pl.ds(dst, 16 * _E)], sem_f).start()
      pltpu.make_async_copy(
          cwl_bpos.at[pl.ds(g * 16, 16)],
          bpos_out.at[pl.ds(base, 16)], sem_f).start()
      return ()

    ngr = (off2 + 15) >> 4
    lax.fori_loop(0, ngr, do_group, ())

    def drain_group(g, _):
      base = pl.multiple_of(w * _SLOT + tot + g * 16, 16)
      dst = pl.multiple_of(base * _E, 16 * _E)
      pltpu.make_async_copy(
          grp.at[pl.ds(g * (16 * _E), 16 * _E)],
          rows_out.at[pl.ds(dst, 16 * _E)], sem_f).wait()
      pltpu.make_async_copy(
          cwl_bpos.at[pl.ds(g * 16, 16)],
          bpos_out.at[pl.ds(base, 16)], sem_f).wait()
      return ()

    lax.fori_loop(0, ngr, drain_group, ())
    return tot + ((off2 + 15) & ~15)

  bufs = (chunk0, chunk1)
  sems = (sem_c0, sem_c1)
  fire(0, bufs[0], sems[0])
  fire(1, bufs[1], sems[1])

  def chunk_pair(kk, carry):
    tot_s, tot_d = carry
    for par in range(2):
      k = 2 * kk + par
      b, sm = bufs[par], sems[par]
      pltpu.make_async_copy(tab_hbm.at[:, pl.ds(0, _CH)], b, sm).wait()
      c = w + 32 * k
      tot_s = process(c, b, wl_idx_s, wl_bpos_s, n_s, tot_s,
                      rows_s_out, bpos_s_out)
      tot_d = process(c, b, wl_idx_d, wl_bpos_d, n_d, tot_d,
                      rows_d_out, bpos_d_out)

      @pl.when(w + 32 * (k + 2) < _NCH)
      def _():
        fire(k + 2, b, sm)
    return tot_s, tot_d

  tot_s, tot_d = lax.fori_loop(0, 15, chunk_pair,
                               (jnp.int32(0), jnp.int32(0)))

  def write_tots(ts, td):
    grp[pl.ds(0, 16)] = jnp.full((16,), ts, jnp.int32).astype(jnp.float32)
    grp[pl.ds(16, 16)] = jnp.full((16,), td, jnp.int32).astype(jnp.float32)
    pltpu.sync_copy(grp.at[pl.ds(0, 8)], wts_out.at[pl.ds(w * 8, 8)])
    pltpu.sync_copy(grp.at[pl.ds(16, 8)], wtd_out.at[pl.ds(w * 8, 8)])

  @pl.when(w < 16)
  def _():
    b, sm = bufs[0], sems[0]
    pltpu.make_async_copy(tab_hbm.at[:, pl.ds(0, _CH)], b, sm).wait()
    c = w + 32 * 30
    ts = process(c, b, wl_idx_s, wl_bpos_s, n_s, tot_s,
                 rows_s_out, bpos_s_out)
    td = process(c, b, wl_idx_d, wl_bpos_d, n_d, tot_d,
                 rows_d_out, bpos_d_out)
    write_tots(ts, td)

  @pl.when(w >= 16)
  def _():
    write_tots(tot_s, tot_d)


def _sc_gather(s_id, d_id, tab_t):
  mesh = plsc.VectorSubcoreMesh(core_axis_name="c", subcore_axis_name="s")
  rows_ty = jax.ShapeDtypeStruct((_W * _SLOT * _E,), jnp.float32)
  bpos_ty = jax.ShapeDtypeStruct((_W * _SLOT,), jnp.int32)
  cnt_ty = jax.ShapeDtypeStruct((_W * 8,), jnp.float32)
  fn = pl.kernel(
      _sc_gather_body,
      out_type=(rows_ty, rows_ty, bpos_ty, bpos_ty, cnt_ty, cnt_ty),
      mesh=mesh,
      compiler_params=pltpu.CompilerParams(needs_layout_passes=False),
      scratch_types=[
          pltpu.VMEM((_E, _CH), jnp.float32),
          pltpu.VMEM((_E, _CH), jnp.float32),
          pltpu.VMEM((_PIECE,), jnp.int32),
          pltpu.VMEM((_CAP,), jnp.int32),
          pltpu.VMEM((_CAP,), jnp.int32),
          pltpu.VMEM((_CAP,), jnp.int32),
          pltpu.VMEM((_CAP,), jnp.int32),
          pltpu.VMEM((_CCAP,), jnp.int32),
          pltpu.VMEM((_CCAP,), jnp.int32),
          pltpu.VMEM((_CCAP * _E,), jnp.float32),
          pltpu.SemaphoreType.DMA,
          pltpu.SemaphoreType.DMA,
          pltpu.SemaphoreType.DMA,
          pltpu.SemaphoreType.DMA,
      ],
  )
  return fn(s_id, d_id, tab_t)


def _sc_scatter_body(rows_s, rows_d, bpos_s, bpos_d, wts, wtd,
                     hs_out, hd_out, rows_v, bpv, cnt_v, sem, sem2):
  w = lax.axis_index("s") * 2 + lax.axis_index("c")
  pltpu.sync_copy(wts.at[pl.ds(w * 8, 8)], cnt_v.at[pl.ds(0, 8)])
  pltpu.sync_copy(wtd.at[pl.ds(w * 8, 8)], cnt_v.at[pl.ds(8, 8)])
  cnts = cnt_v[pl.ds(0, 16)].astype(jnp.int32)
  for st, (rows, bpos, out, hs_half) in enumerate(
      ((rows_s, bpos_s, hs_out, 0), (rows_d, bpos_d, hd_out, 1))):
    ng = (cnts[8 * st] + 127) >> 7

    def stage_and_fire(j, _):
      pltpu.make_async_copy(
          rows.at[pl.ds(w * _SLOT + j * 128, 128)],
          rows_v.at[st * (_SLOT // 128) + j], sem2).start()
      pltpu.make_async_copy(
          bpos.at[pl.ds(w * _SLOT + j * 128, 128)],
          bpv.at[st * (_SLOT // 128) + j], sem2).start()
      pltpu.make_async_copy(
          rows.at[pl.ds(w * _SLOT + j * 128, 128)],
          rows_v.at[st * (_SLOT // 128) + j], sem2).wait()
      pltpu.make_async_copy(
          bpos.at[pl.ds(w * _SLOT + j * 128, 128)],
          bpv.at[st * (_SLOT // 128) + j], sem2).wait()
      pltpu.make_async_copy(
          rows_v.at[st * (_SLOT // 128) + j],
          out.at[bpv.at[st * (_SLOT // 128) + j]], sem).start()
      return ()

    lax.fori_loop(0, ng, stage_and_fire, ())

    def drain(j, _):
      pltpu.make_async_copy(
          rows_v.at[st * (_SLOT // 128) + j],
          out.at[bpv.at[st * (_SLOT // 128) + j]], sem).wait()
      return ()

    lax.fori_loop(0, ng, drain, ())


def _sc_scatter(rows_s, rows_d, bpos_s, bpos_d, wts, wtd):
  mesh = plsc.VectorSubcoreMesh(core_axis_name="c", subcore_axis_name="s")
  out_ty = jax.ShapeDtypeStruct((_B + 16, _E), jnp.float32)
  fn = pl.kernel(
      _sc_scatter_body,
      out_type=(out_ty, out_ty),
      mesh=mesh,
      compiler_params=pltpu.CompilerParams(use_tc_tiling_on_sc=False,
                                           needs_layout_passes=False),
      scratch_types=[
          pltpu.VMEM((2 * (_SLOT // 128), 128, _E), jnp.float32),
          pltpu.VMEM((2 * (_SLOT // 128), 128), jnp.int32),
          pltpu.VMEM((16,), jnp.float32),
          pltpu.SemaphoreType.DMA,
          pltpu.SemaphoreType.DMA,
      ],
  )
  return fn(rows_s.reshape(_W * _SLOT, _E), rows_d.reshape(_W * _SLOT, _E),
            bpos_s, bpos_d, wts, wtd)


def _tc_compute_body(hs_ref, hd_ref, q_ref, sid_ref, did_ref, tail_ref,
                     out_ref):
  it = lax.broadcasted_iota(jnp.int32, (_BLK, 640), 1)

  def patch(h, ids):
    os = ids[...] - _TAIL
    oh = (os == it).astype(jnp.float32)
    fix = jnp.dot(oh, tail_ref[...], preferred_element_type=jnp.float32)
    return jnp.where(os >= 0, fix, h[...])

  hs = patch(hs_ref, sid_ref)
  hd = patch(hd_ref, did_ref)
  t = jnp.dot(hs, q_ref[...], preferred_element_type=jnp.float32)
  out_ref[...] = jnp.exp(jnp.sum(t * hd, axis=1))


def _tc_compute(h_s, h_d, Q, s_id, d_id, h_tail):
  grid = _B // _BLK
  return pl.pallas_call(
      _tc_compute_body,
      grid=(grid,),
      in_specs=[
          pl.BlockSpec((_BLK, _E), lambda i: (i, 0)),
          pl.BlockSpec((_BLK, _E), lambda i: (i, 0)),
          pl.BlockSpec((_E, _E), lambda i: (0, 0)),
          pl.BlockSpec((_BLK, 1), lambda i: (i, 0)),
          pl.BlockSpec((_BLK, 1), lambda i: (i, 0)),
          pl.BlockSpec((640, _E), lambda i: (0, 0)),
      ],
      out_specs=pl.BlockSpec((_BLK,), lambda i: (i,)),
      out_shape=jax.ShapeDtypeStruct((_B,), jnp.float32),
  )(h_s, h_d, Q, s_id.reshape(_B, 1), d_id.reshape(_B, 1), h_tail)


@jax.jit
def kernel(s_id, d_id, h_static, Q):
  s_id = s_id.astype(jnp.int32)
  d_id = d_id.astype(jnp.int32)
  tab_t = h_static.T
  rows_s, rows_d, bpos_s, bpos_d, wts, wtd = _sc_gather(s_id, d_id, tab_t)
  hs_pad, hd_pad = _sc_scatter(rows_s, rows_d, bpos_s, bpos_d, wts, wtd)
  h_tail = jnp.zeros((640, _E), jnp.float32).at[: _N - _TAIL].set(
      h_static[_TAIL:])
  return _tc_compute(hs_pad[:_B], hd_pad[:_B], Q, s_id, d_id, h_tail)


# speedup vs baseline: 3.3607x; 1.1047x over previous
import functools

import jax
import jax.numpy as jnp
from jax import lax
from jax.experimental import pallas as pl
from jax.experimental.pallas import tpu as pltpu
from jax.experimental.pallas import tpu_sc as plsc

_N = 1000001
_E = 32
_B = 16384
_W = 32
_CH = 1024
_NCH = 976
_TAIL = 999424
_CAP = 784
_SLOT = 1536
_CCAP = 272
_PIECE = 2048
_DEAD = 1 << 20
_BLK = 2048


def _iota16():
  return lax.iota(jnp.int32, 16)


def _sc_gather_body(s_hbm, d_hbm, tab_hbm,
                    rows_s_out, rows_d_out, bpos_s_out, bpos_d_out,
                    wts_out, wtd_out,
                    chunk0, chunk1, piece_s, piece_d,
                    wl_idx_s, wl_bpos_s, wl_idx_d, wl_bpos_d,
                    cwl_loc_s, cwl_bpos_s, cwl_loc_d, cwl_bpos_d,
                    grp_s, grp_d,
                    sem_p, sem_c0, sem_c1, sem_f):
  w = lax.axis_index("s") * 2 + lax.axis_index("c")
  it16 = _iota16()

  sent_bp = _B + it16
  dead = jnp.full((16,), _DEAD, jnp.int32)
  for v in range(_CAP // 16):
    wl_bpos_s[pl.ds(v * 16, 16)] = sent_bp
    wl_idx_s[pl.ds(v * 16, 16)] = dead
    wl_idx_d[pl.ds(v * 16, 16)] = dead
  for v in range(_CCAP // 16):
    cwl_loc_s[pl.ds(v * 16, 16)] = jnp.zeros((16,), jnp.int32)
    cwl_loc_d[pl.ds(v * 16, 16)] = jnp.zeros((16,), jnp.int32)
  for h in range(2):
    pltpu.sync_copy(wl_bpos_s.at[pl.ds(0, _SLOT // 2)],
                    bpos_s_out.at[pl.ds(w * _SLOT + h * (_SLOT // 2),
                                        _SLOT // 2)])
    pltpu.sync_copy(wl_bpos_s.at[pl.ds(0, _SLOT // 2)],
                    bpos_d_out.at[pl.ds(w * _SLOT + h * (_SLOT // 2),
                                        _SLOT // 2)])
  for v in range(_CAP // 16):
    wl_bpos_d[pl.ds(v * 16, 16)] = sent_bp

  def fire(k, buf, sem):
    c = w + 32 * k
    pltpu.make_async_copy(
        tab_hbm.at[:, pl.ds(c * _CH, _CH)], buf, sem).start()

  fire(0, chunk0, sem_c0)
  fire(1, chunk1, sem_c1)

  zero_v = jnp.zeros((16,), jnp.int32)
  off_s = zero_v
  off_d = zero_v
  for p in range(_B // _PIECE):
    pltpu.sync_copy(s_hbm.at[pl.ds(p * _PIECE, _PIECE)], piece_s)
    pltpu.sync_copy(d_hbm.at[pl.ds(p * _PIECE, _PIECE)], piece_d)

    def scan_vec(v, carry):
      o_s, o_d = carry
      idx_s = piece_s[pl.ds(v * 16, 16)]
      idx_d = piece_d[pl.ds(v * 16, 16)]
      m_s = ((idx_s >> 10) & 31) == w
      m_d = ((idx_d >> 10) & 31) == w
      cs_s = plsc.cumsum(m_s.astype(jnp.int32))
      cs_d = plsc.cumsum(m_d.astype(jnp.int32))
      pos_s = o_s + cs_s - 1
      pos_d = o_d + cs_d - 1
      bp = jnp.full((16,), p * _PIECE, jnp.int32) + v * 16 + it16
      plsc.store_scatter(wl_idx_s, [pos_s], idx_s, mask=m_s)
      plsc.store_scatter(wl_bpos_s, [pos_s], bp, mask=m_s)
      plsc.store_scatter(wl_idx_d, [pos_d], idx_d, mask=m_d)
      plsc.store_scatter(wl_bpos_d, [pos_d], bp, mask=m_d)
      n_s = plsc.all_reduce_population_count(m_s)
      n_d = plsc.all_reduce_population_count(m_d)
      return o_s + n_s, o_d + n_d

    off_s, off_d = lax.fori_loop(0, _PIECE // 16, scan_vec, (off_s, off_d))
  n_s = off_s[0]
  n_d = off_d[0]

  def process(c, buf, tot_s, tot_d):
    for v in range(_CCAP // 16):
      cwl_bpos_s[pl.ds(v * 16, 16)] = sent_bp
      cwl_bpos_d[pl.ds(v * 16, 16)] = sent_bp

    def scan_wl(v, carry):
      o_s, o_d = carry
      wi_s = wl_idx_s[pl.ds(v * 16, 16)]
      wi_d = wl_idx_d[pl.ds(v * 16, 16)]
      m_s = (wi_s >> 10) == c
      m_d = (wi_d >> 10) == c
      cs_s = plsc.cumsum(m_s.astype(jnp.int32))
      cs_d = plsc.cumsum(m_d.astype(jnp.int32))
      pos_s = o_s + cs_s - 1
      pos_d = o_d + cs_d - 1
      plsc.store_scatter(cwl_loc_s, [pos_s], wi_s & 1023, mask=m_s)
      plsc.store_scatter(cwl_loc_d, [pos_d], wi_d & 1023, mask=m_d)
      bp_s = wl_bpos_s[pl.ds(v * 16, 16)]
      bp_d = wl_bpos_d[pl.ds(v * 16, 16)]
      plsc.store_scatter(cwl_bpos_s, [pos_s], bp_s, mask=m_s)
      plsc.store_scatter(cwl_bpos_d, [pos_d], bp_d, mask=m_d)
      return (o_s + plsc.all_reduce_population_count(m_s),
              o_d + plsc.all_reduce_population_count(m_d))

    n_vecs = (jnp.maximum(n_s, n_d) + 15) >> 4
    off2_s, off2_d = lax.fori_loop(0, n_vecs, scan_wl, (zero_v, zero_v))

    def make_group(grp, cwl_loc, cwl_bpos, rows_out, bpos_out, tot):
      def do_group(g, _):
        base = pl.multiple_of(w * _SLOT + tot + g * 16, 16)
        loc16 = cwl_loc[pl.ds(g * 16, 16)]
        for j in range(_E):
          vals = plsc.load_gather(
              buf, [jnp.full((16,), j, jnp.int32), loc16])
          plsc.store_scatter(grp, [g * (16 * _E) + it16 * _E + j], vals)
        dst = pl.multiple_of(base * _E, 16 * _E)
        pltpu.make_async_copy(
            grp.at[pl.ds(g * (16 * _E), 16 * _E)],
            rows_out.at[pl.ds(dst, 16 * _E)], sem_f).start()
        pltpu.make_async_copy(
            cwl_bpos.at[pl.ds(g * 16, 16)],
            bpos_out.at[pl.ds(base, 16)], sem_f).start()
        return ()

      def drain_group(g, _):
        base = pl.multiple_of(w * _SLOT + tot + g * 16, 16)
        dst = pl.multiple_of(base * _E, 16 * _E)
        pltpu.make_async_copy(
            grp.at[pl.ds(g * (16 * _E), 16 * _E)],
            rows_out.at[pl.ds(dst, 16 * _E)], sem_f).wait()
        pltpu.make_async_copy(
            cwl_bpos.at[pl.ds(g * 16, 16)],
            bpos_out.at[pl.ds(base, 16)], sem_f).wait()
        return ()

      return do_group, drain_group

    og_s = off2_s[0]
    og_d = off2_d[0]
    do_s, dr_s = make_group(grp_s, cwl_loc_s, cwl_bpos_s,
                            rows_s_out, bpos_s_out, tot_s)
    do_d, dr_d = make_group(grp_d, cwl_loc_d, cwl_bpos_d,
                            rows_d_out, bpos_d_out, tot_d)
    ngr_s = (og_s + 15) >> 4
    ngr_d = (og_d + 15) >> 4
    lax.fori_loop(0, ngr_s, do_s, ())
    lax.fori_loop(0, ngr_d, do_d, ())
    lax.fori_loop(0, ngr_s, dr_s, ())
    lax.fori_loop(0, ngr_d, dr_d, ())
    return tot_s + ((og_s + 15) & ~15), tot_d + ((og_d + 15) & ~15)

  bufs = (chunk0, chunk1)
  sems = (sem_c0, sem_c1)

  def chunk_pair(kk, carry):
    tot_s, tot_d = carry
    for par in range(2):
      k = 2 * kk + par
      b, sm = bufs[par], sems[par]
      pltpu.make_async_copy(tab_hbm.at[:, pl.ds(0, _CH)], b, sm).wait()
      c = w + 32 * k
      tot_s, tot_d = process(c, b, tot_s, tot_d)

      @pl.when(w + 32 * (k + 2) < _NCH)
      def _():
        fire(k + 2, b, sm)
    return tot_s, tot_d

  tot_s, tot_d = lax.fori_loop(0, 15, chunk_pair,
                               (jnp.int32(0), jnp.int32(0)))

  def write_tots(ts, td):
    grp_s[pl.ds(0, 16)] = jnp.full((16,), ts, jnp.int32).astype(jnp.float32)
    grp_s[pl.ds(16, 16)] = jnp.full((16,), td, jnp.int32).astype(jnp.float32)
    pltpu.sync_copy(grp_s.at[pl.ds(0, 8)], wts_out.at[pl.ds(w * 8, 8)])
    pltpu.sync_copy(grp_s.at[pl.ds(16, 8)], wtd_out.at[pl.ds(w * 8, 8)])

  @pl.when(w < 16)
  def _():
    b, sm = bufs[0], sems[0]
    pltpu.make_async_copy(tab_hbm.at[:, pl.ds(0, _CH)], b, sm).wait()
    c = w + 32 * 30
    ts, td = process(c, b, tot_s, tot_d)
    write_tots(ts, td)

  @pl.when(w >= 16)
  def _():
    write_tots(tot_s, tot_d)


def _sc_gather(s_id, d_id, tab_t):
  mesh = plsc.VectorSubcoreMesh(core_axis_name="c", subcore_axis_name="s")
  rows_ty = jax.ShapeDtypeStruct((_W * _SLOT * _E,), jnp.float32)
  bpos_ty = jax.ShapeDtypeStruct((_W * _SLOT,), jnp.int32)
  cnt_ty = jax.ShapeDtypeStruct((_W * 8,), jnp.float32)
  fn = pl.kernel(
      _sc_gather_body,
      out_type=(rows_ty, rows_ty, bpos_ty, bpos_ty, cnt_ty, cnt_ty),
      mesh=mesh,
      compiler_params=pltpu.CompilerParams(needs_layout_passes=False),
      scratch_types=[
          pltpu.VMEM((_E, _CH), jnp.float32),
          pltpu.VMEM((_E, _CH), jnp.float32),
          pltpu.VMEM((_PIECE,), jnp.int32),
          pltpu.VMEM((_PIECE,), jnp.int32),
          pltpu.VMEM((_CAP,), jnp.int32),
          pltpu.VMEM((_CAP,), jnp.int32),
          pltpu.VMEM((_CAP,), jnp.int32),
          pltpu.VMEM((_CAP,), jnp.int32),
          pltpu.VMEM((_CCAP,), jnp.int32),
          pltpu.VMEM((_CCAP,), jnp.int32),
          pltpu.VMEM((_CCAP,), jnp.int32),
          pltpu.VMEM((_CCAP,), jnp.int32),
          pltpu.VMEM((_CCAP * _E,), jnp.float32),
          pltpu.VMEM((_CCAP * _E,), jnp.float32),
          pltpu.SemaphoreType.DMA,
          pltpu.SemaphoreType.DMA,
          pltpu.SemaphoreType.DMA,
          pltpu.SemaphoreType.DMA,
      ],
  )
  return fn(s_id, d_id, tab_t)


def _sc_scatter_body(rows_s, rows_d, bpos_s, bpos_d, wts, wtd,
                     hs_out, hd_out, rows_v, bpv, cnt_v, sem, sem2):
  w = lax.axis_index("s") * 2 + lax.axis_index("c")
  pltpu.sync_copy(wts.at[pl.ds(w * 8, 8)], cnt_v.at[pl.ds(0, 8)])
  pltpu.sync_copy(wtd.at[pl.ds(w * 8, 8)], cnt_v.at[pl.ds(8, 8)])
  cnts = cnt_v[pl.ds(0, 16)].astype(jnp.int32)
  for st, (rows, bpos, out, hs_half) in enumerate(
      ((rows_s, bpos_s, hs_out, 0), (rows_d, bpos_d, hd_out, 1))):
    ng = (cnts[8 * st] + 127) >> 7

    def stage_and_fire(j, _):
      pltpu.make_async_copy(
          rows.at[pl.ds(w * _SLOT + j * 128, 128)],
          rows_v.at[st * (_SLOT // 128) + j], sem2).start()
      pltpu.make_async_copy(
          bpos.at[pl.ds(w * _SLOT + j * 128, 128)],
          bpv.at[st * (_SLOT // 128) + j], sem2).start()
      pltpu.make_async_copy(
          rows.at[pl.ds(w * _SLOT + j * 128, 128)],
          rows_v.at[st * (_SLOT // 128) + j], sem2).wait()
      pltpu.make_async_copy(
          bpos.at[pl.ds(w * _SLOT + j * 128, 128)],
          bpv.at[st * (_SLOT // 128) + j], sem2).wait()
      pltpu.make_async_copy(
          rows_v.at[st * (_SLOT // 128) + j],
          out.at[bpv.at[st * (_SLOT // 128) + j]], sem).start()
      return ()

    lax.fori_loop(0, ng, stage_and_fire, ())

    def drain(j, _):
      pltpu.make_async_copy(
          rows_v.at[st * (_SLOT // 128) + j],
          out.at[bpv.at[st * (_SLOT // 128) + j]], sem).wait()
      return ()

    lax.fori_loop(0, ng, drain, ())


def _sc_scatter(rows_s, rows_d, bpos_s, bpos_d, wts, wtd):
  mesh = plsc.VectorSubcoreMesh(core_axis_name="c", subcore_axis_name="s")
  out_ty = jax.ShapeDtypeStruct((_B + 16, _E), jnp.float32)
  fn = pl.kernel(
      _sc_scatter_body,
      out_type=(out_ty, out_ty),
      mesh=mesh,
      compiler_params=pltpu.CompilerParams(use_tc_tiling_on_sc=False,
                                           needs_layout_passes=False),
      scratch_types=[
          pltpu.VMEM((2 * (_SLOT // 128), 128, _E), jnp.float32),
          pltpu.VMEM((2 * (_SLOT // 128), 128), jnp.int32),
          pltpu.VMEM((16,), jnp.float32),
          pltpu.SemaphoreType.DMA,
          pltpu.SemaphoreType.DMA,
      ],
  )
  return fn(rows_s.reshape(_W * _SLOT, _E), rows_d.reshape(_W * _SLOT, _E),
            bpos_s, bpos_d, wts, wtd)


def _tc_compute_body(hs_ref, hd_ref, q_ref, sid_ref, did_ref, tail_ref,
                     out_ref):
  it = lax.broadcasted_iota(jnp.int32, (_BLK, 640), 1)

  def patch(h, ids):
    os = ids[...] - _TAIL
    oh = (os == it).astype(jnp.float32)
    fix = jnp.dot(oh, tail_ref[...], preferred_element_type=jnp.float32)
    return jnp.where(os >= 0, fix, h[...])

  hs = patch(hs_ref, sid_ref)
  hd = patch(hd_ref, did_ref)
  t = jnp.dot(hs, q_ref[...], preferred_element_type=jnp.float32)
  out_ref[...] = jnp.exp(jnp.sum(t * hd, axis=1))


def _tc_compute(h_s, h_d, Q, s_id, d_id, h_tail):
  grid = _B // _BLK
  return pl.pallas_call(
      _tc_compute_body,
      grid=(grid,),
      in_specs=[
          pl.BlockSpec((_BLK, _E), lambda i: (i, 0)),
          pl.BlockSpec((_BLK, _E), lambda i: (i, 0)),
          pl.BlockSpec((_E, _E), lambda i: (0, 0)),
          pl.BlockSpec((_BLK, 1), lambda i: (i, 0)),
          pl.BlockSpec((_BLK, 1), lambda i: (i, 0)),
          pl.BlockSpec((640, _E), lambda i: (0, 0)),
      ],
      out_specs=pl.BlockSpec((_BLK,), lambda i: (i,)),
      out_shape=jax.ShapeDtypeStruct((_B,), jnp.float32),
  )(h_s, h_d, Q, s_id.reshape(_B, 1), d_id.reshape(_B, 1), h_tail)


@jax.jit
def kernel(s_id, d_id, h_static, Q):
  s_id = s_id.astype(jnp.int32)
  d_id = d_id.astype(jnp.int32)
  tab_t = h_static.T
  rows_s, rows_d, bpos_s, bpos_d, wts, wtd = _sc_gather(s_id, d_id, tab_t)
  hs_pad, hd_pad = _sc_scatter(rows_s, rows_d, bpos_s, bpos_d, wts, wtd)
  h_tail = jnp.zeros((640, _E), jnp.float32).at[: _N - _TAIL].set(
      h_static[_TAIL:])
  return _tc_compute(hs_pad[:_B], hd_pad[:_B], Q, s_id, d_id, h_tail)


# speedup vs baseline: 3.6135x; 1.0752x over previous
import functools

import jax
import jax.numpy as jnp
from jax import lax
from jax.experimental import pallas as pl
from jax.experimental.pallas import tpu as pltpu
from jax.experimental.pallas import tpu_sc as plsc

_N = 1000001
_E = 32
_B = 16384
_W = 32
_CH = 1024
_NCH = 976
_TAIL = 999936
_CAP = 784
_SLOT = 1536
_CCAP = 272
_PIECE = 2048
_DEAD = 1 << 20
_BLK = 2048


def _iota16():
  return lax.iota(jnp.int32, 16)


def _sc_gather_body(s_hbm, d_hbm, tab_hbm,
                    rows_s_out, rows_d_out, bpos_s_out, bpos_d_out,
                    wts_out, wtd_out,
                    chunk0, chunk1, piece_s, piece_d,
                    wl_idx_s, wl_bpos_s, wl_idx_d, wl_bpos_d,
                    cwl_loc_s, cwl_bpos_s, cwl_loc_d, cwl_bpos_d,
                    grp_s, grp_d,
                    sem_p, sem_c0, sem_c1, sem_f):
  w = lax.axis_index("s") * 2 + lax.axis_index("c")
  it16 = _iota16()

  sent_bp = _B + it16
  dead = jnp.full((16,), _DEAD, jnp.int32)
  for v in range(_CAP // 16):
    wl_bpos_s[pl.ds(v * 16, 16)] = sent_bp
    wl_idx_s[pl.ds(v * 16, 16)] = dead
    wl_idx_d[pl.ds(v * 16, 16)] = dead
  for v in range(_CCAP // 16):
    cwl_loc_s[pl.ds(v * 16, 16)] = jnp.zeros((16,), jnp.int32)
    cwl_loc_d[pl.ds(v * 16, 16)] = jnp.zeros((16,), jnp.int32)
  for h in range(2):
    pltpu.sync_copy(wl_bpos_s.at[pl.ds(0, _SLOT // 2)],
                    bpos_s_out.at[pl.ds(w * _SLOT + h * (_SLOT // 2),
                                        _SLOT // 2)])
    pltpu.sync_copy(wl_bpos_s.at[pl.ds(0, _SLOT // 2)],
                    bpos_d_out.at[pl.ds(w * _SLOT + h * (_SLOT // 2),
                                        _SLOT // 2)])
  for v in range(_CAP // 16):
    wl_bpos_d[pl.ds(v * 16, 16)] = sent_bp

  def fire(k, buf, sem):
    c = w + 32 * k
    pltpu.make_async_copy(
        tab_hbm.at[:, pl.ds(c * _CH, _CH)], buf, sem).start()

  fire(0, chunk0, sem_c0)
  fire(1, chunk1, sem_c1)

  zero_v = jnp.zeros((16,), jnp.int32)
  off_s = zero_v
  off_d = zero_v
  for p in range(_B // _PIECE):
    pltpu.sync_copy(s_hbm.at[pl.ds(p * _PIECE, _PIECE)], piece_s)
    pltpu.sync_copy(d_hbm.at[pl.ds(p * _PIECE, _PIECE)], piece_d)

    def scan_vec(v, carry):
      o_s, o_d = carry
      idx_s = piece_s[pl.ds(v * 16, 16)]
      idx_d = piece_d[pl.ds(v * 16, 16)]
      m_s = ((idx_s >> 10) & 31) == w
      m_d = ((idx_d >> 10) & 31) == w
      cs_s = plsc.cumsum(m_s.astype(jnp.int32))
      cs_d = plsc.cumsum(m_d.astype(jnp.int32))
      pos_s = o_s + cs_s - 1
      pos_d = o_d + cs_d - 1
      bp = jnp.full((16,), p * _PIECE, jnp.int32) + v * 16 + it16
      plsc.store_scatter(wl_idx_s, [pos_s], idx_s, mask=m_s)
      plsc.store_scatter(wl_bpos_s, [pos_s], bp, mask=m_s)
      plsc.store_scatter(wl_idx_d, [pos_d], idx_d, mask=m_d)
      plsc.store_scatter(wl_bpos_d, [pos_d], bp, mask=m_d)
      n_s = plsc.all_reduce_population_count(m_s)
      n_d = plsc.all_reduce_population_count(m_d)
      return o_s + n_s, o_d + n_d

    off_s, off_d = lax.fori_loop(0, _PIECE // 16, scan_vec, (off_s, off_d))
  n_s = off_s[0]
  n_d = off_d[0]

  def process(c, buf, tot_s, tot_d):
    for v in range(_CCAP // 16):
      cwl_bpos_s[pl.ds(v * 16, 16)] = sent_bp
      cwl_bpos_d[pl.ds(v * 16, 16)] = sent_bp

    def scan_wl(v, carry):
      o_s, o_d = carry
      wi_s = wl_idx_s[pl.ds(v * 16, 16)]
      wi_d = wl_idx_d[pl.ds(v * 16, 16)]
      m_s = (wi_s >> 10) == c
      m_d = (wi_d >> 10) == c
      cs_s = plsc.cumsum(m_s.astype(jnp.int32))
      cs_d = plsc.cumsum(m_d.astype(jnp.int32))
      pos_s = o_s + cs_s - 1
      pos_d = o_d + cs_d - 1
      plsc.store_scatter(cwl_loc_s, [pos_s], wi_s & 1023, mask=m_s)
      plsc.store_scatter(cwl_loc_d, [pos_d], wi_d & 1023, mask=m_d)
      bp_s = wl_bpos_s[pl.ds(v * 16, 16)]
      bp_d = wl_bpos_d[pl.ds(v * 16, 16)]
      plsc.store_scatter(cwl_bpos_s, [pos_s], bp_s, mask=m_s)
      plsc.store_scatter(cwl_bpos_d, [pos_d], bp_d, mask=m_d)
      return (o_s + plsc.all_reduce_population_count(m_s),
              o_d + plsc.all_reduce_population_count(m_d))

    n_vecs = (jnp.maximum(n_s, n_d) + 15) >> 4
    off2_s, off2_d = lax.fori_loop(0, n_vecs, scan_wl, (zero_v, zero_v))

    def make_group(grp, cwl_loc, cwl_bpos, rows_out, bpos_out, tot):
      def do_group(g, _):
        base = pl.multiple_of(w * _SLOT + tot + g * 16, 16)
        loc16 = cwl_loc[pl.ds(g * 16, 16)]
        for j in range(_E):
          vals = plsc.load_gather(
              buf, [jnp.full((16,), j, jnp.int32), loc16])
          plsc.store_scatter(grp, [g * (16 * _E) + it16 * _E + j], vals)
        dst = pl.multiple_of(base * _E, 16 * _E)
        pltpu.make_async_copy(
            grp.at[pl.ds(g * (16 * _E), 16 * _E)],
            rows_out.at[pl.ds(dst, 16 * _E)], sem_f).start()
        pltpu.make_async_copy(
            cwl_bpos.at[pl.ds(g * 16, 16)],
            bpos_out.at[pl.ds(base, 16)], sem_f).start()
        return ()

      def drain_group(g, _):
        base = pl.multiple_of(w * _SLOT + tot + g * 16, 16)
        dst = pl.multiple_of(base * _E, 16 * _E)
        pltpu.make_async_copy(
            grp.at[pl.ds(g * (16 * _E), 16 * _E)],
            rows_out.at[pl.ds(dst, 16 * _E)], sem_f).wait()
        pltpu.make_async_copy(
            cwl_bpos.at[pl.ds(g * 16, 16)],
            bpos_out.at[pl.ds(base, 16)], sem_f).wait()
        return ()

      return do_group, drain_group

    og_s = off2_s[0]
    og_d = off2_d[0]
    do_s, dr_s = make_group(grp_s, cwl_loc_s, cwl_bpos_s,
                            rows_s_out, bpos_s_out, tot_s)
    do_d, dr_d = make_group(grp_d, cwl_loc_d, cwl_bpos_d,
                            rows_d_out, bpos_d_out, tot_d)
    ngr_s = (og_s + 15) >> 4
    ngr_d = (og_d + 15) >> 4
    lax.fori_loop(0, ngr_s, do_s, ())
    lax.fori_loop(0, ngr_d, do_d, ())
    lax.fori_loop(0, ngr_s, dr_s, ())
    lax.fori_loop(0, ngr_d, dr_d, ())
    return tot_s + ((og_s + 15) & ~15), tot_d + ((og_d + 15) & ~15)

  bufs = (chunk0, chunk1)
  sems = (sem_c0, sem_c1)

  def chunk_pair(kk, carry):
    tot_s, tot_d = carry
    for par in range(2):
      k = 2 * kk + par
      b, sm = bufs[par], sems[par]
      pltpu.make_async_copy(tab_hbm.at[:, pl.ds(0, _CH)], b, sm).wait()
      c = w + 32 * k
      tot_s, tot_d = process(c, b, tot_s, tot_d)

      @pl.when(w + 32 * (k + 2) < _NCH)
      def _():
        fire(k + 2, b, sm)
    return tot_s, tot_d

  tot_s, tot_d = lax.fori_loop(0, 15, chunk_pair,
                               (jnp.int32(0), jnp.int32(0)))

  def write_tots(ts, td):
    grp_s[pl.ds(0, 16)] = jnp.full((16,), ts, jnp.int32).astype(jnp.float32)
    grp_s[pl.ds(16, 16)] = jnp.full((16,), td, jnp.int32).astype(jnp.float32)
    pltpu.sync_copy(grp_s.at[pl.ds(0, 8)], wts_out.at[pl.ds(w * 8, 8)])
    pltpu.sync_copy(grp_s.at[pl.ds(16, 8)], wtd_out.at[pl.ds(w * 8, 8)])

  @pl.when(w < 16)
  def _():
    b, sm = bufs[0], sems[0]
    pltpu.make_async_copy(tab_hbm.at[:, pl.ds(0, _CH)], b, sm).wait()
    c = w + 32 * 30
    ts, td = process(c, b, tot_s, tot_d)
    write_tots(ts, td)

  @pl.when(w == 16)
  def _():
    b, sm = bufs[0], sems[0]
    pltpu.make_async_copy(
        tab_hbm.at[:, pl.ds(976 * _CH, 512)],
        b.at[:, pl.ds(0, 512)], sm).start()
    pltpu.make_async_copy(
        tab_hbm.at[:, pl.ds(0, 512)], b.at[:, pl.ds(0, 512)], sm).wait()
    ts, td = process(jnp.int32(976), b, tot_s, tot_d)
    write_tots(ts, td)

  @pl.when(w > 16)
  def _():
    write_tots(tot_s, tot_d)


def _sc_gather(s_id, d_id, tab_t):
  mesh = plsc.VectorSubcoreMesh(core_axis_name="c", subcore_axis_name="s")
  rows_ty = jax.ShapeDtypeStruct((_W * _SLOT * _E,), jnp.float32)
  bpos_ty = jax.ShapeDtypeStruct((_W * _SLOT,), jnp.int32)
  cnt_ty = jax.ShapeDtypeStruct((_W * 8,), jnp.float32)
  fn = pl.kernel(
      _sc_gather_body,
      out_type=(rows_ty, rows_ty, bpos_ty, bpos_ty, cnt_ty, cnt_ty),
      mesh=mesh,
      compiler_params=pltpu.CompilerParams(needs_layout_passes=False),
      scratch_types=[
          pltpu.VMEM((_E, _CH), jnp.float32),
          pltpu.VMEM((_E, _CH), jnp.float32),
          pltpu.VMEM((_PIECE,), jnp.int32),
          pltpu.VMEM((_PIECE,), jnp.int32),
          pltpu.VMEM((_CAP,), jnp.int32),
          pltpu.VMEM((_CAP,), jnp.int32),
          pltpu.VMEM((_CAP,), jnp.int32),
          pltpu.VMEM((_CAP,), jnp.int32),
          pltpu.VMEM((_CCAP,), jnp.int32),
          pltpu.VMEM((_CCAP,), jnp.int32),
          pltpu.VMEM((_CCAP,), jnp.int32),
          pltpu.VMEM((_CCAP,), jnp.int32),
          pltpu.VMEM((_CCAP * _E,), jnp.float32),
          pltpu.VMEM((_CCAP * _E,), jnp.float32),
          pltpu.SemaphoreType.DMA,
          pltpu.SemaphoreType.DMA,
          pltpu.SemaphoreType.DMA,
          pltpu.SemaphoreType.DMA,
      ],
  )
  return fn(s_id, d_id, tab_t)


def _sc_scatter_body(rows_s, rows_d, bpos_s, bpos_d, wts, wtd,
                     hs_out, hd_out, rows_v, bpv, cnt_v, sem, sem2):
  w = lax.axis_index("s") * 2 + lax.axis_index("c")
  pltpu.sync_copy(wts.at[pl.ds(w * 8, 8)], cnt_v.at[pl.ds(0, 8)])
  pltpu.sync_copy(wtd.at[pl.ds(w * 8, 8)], cnt_v.at[pl.ds(8, 8)])
  cnts = cnt_v[pl.ds(0, 16)].astype(jnp.int32)
  for st, (rows, bpos, out, hs_half) in enumerate(
      ((rows_s, bpos_s, hs_out, 0), (rows_d, bpos_d, hd_out, 1))):
    ng = (cnts[8 * st] + 127) >> 7

    def stage_and_fire(j, _):
      pltpu.make_async_copy(
          rows.at[pl.ds(w * _SLOT + j * 128, 128)],
          rows_v.at[st * (_SLOT // 128) + j], sem2).start()
      pltpu.make_async_copy(
          bpos.at[pl.ds(w * _SLOT + j * 128, 128)],
          bpv.at[st * (_SLOT // 128) + j], sem2).start()
      pltpu.make_async_copy(
          rows.at[pl.ds(w * _SLOT + j * 128, 128)],
          rows_v.at[st * (_SLOT // 128) + j], sem2).wait()
      pltpu.make_async_copy(
          bpos.at[pl.ds(w * _SLOT + j * 128, 128)],
          bpv.at[st * (_SLOT // 128) + j], sem2).wait()
      pltpu.make_async_copy(
          rows_v.at[st * (_SLOT // 128) + j],
          out.at[bpv.at[st * (_SLOT // 128) + j]], sem).start()
      return ()

    lax.fori_loop(0, ng, stage_and_fire, ())

    def drain(j, _):
      pltpu.make_async_copy(
          rows_v.at[st * (_SLOT // 128) + j],
          out.at[bpv.at[st * (_SLOT // 128) + j]], sem).wait()
      return ()

    lax.fori_loop(0, ng, drain, ())


def _sc_scatter(rows_s, rows_d, bpos_s, bpos_d, wts, wtd):
  mesh = plsc.VectorSubcoreMesh(core_axis_name="c", subcore_axis_name="s")
  out_ty = jax.ShapeDtypeStruct((_B + 16, _E), jnp.float32)
  fn = pl.kernel(
      _sc_scatter_body,
      out_type=(out_ty, out_ty),
      mesh=mesh,
      compiler_params=pltpu.CompilerParams(use_tc_tiling_on_sc=False,
                                           needs_layout_passes=False),
      scratch_types=[
          pltpu.VMEM((2 * (_SLOT // 128), 128, _E), jnp.float32),
          pltpu.VMEM((2 * (_SLOT // 128), 128), jnp.int32),
          pltpu.VMEM((16,), jnp.float32),
          pltpu.SemaphoreType.DMA,
          pltpu.SemaphoreType.DMA,
      ],
  )
  return fn(rows_s.reshape(_W * _SLOT, _E), rows_d.reshape(_W * _SLOT, _E),
            bpos_s, bpos_d, wts, wtd)


def _tc_compute_body(hs_ref, hd_ref, q_ref, sid_ref, did_ref, tail_ref,
                     out_ref):
  it = lax.broadcasted_iota(jnp.int32, (_BLK, 128), 1)

  def patch(h, ids):
    os = ids[...] - _TAIL
    oh = (os == it).astype(jnp.float32)
    fix = jnp.dot(oh, tail_ref[...], preferred_element_type=jnp.float32)
    return jnp.where(os >= 0, fix, h[...])

  hs = patch(hs_ref, sid_ref)
  hd = patch(hd_ref, did_ref)
  t = jnp.dot(hs, q_ref[...], preferred_element_type=jnp.float32)
  out_ref[...] = jnp.exp(jnp.sum(t * hd, axis=1))


def _tc_compute(h_s, h_d, Q, s_id, d_id, h_tail):
  grid = _B // _BLK
  return pl.pallas_call(
      _tc_compute_body,
      grid=(grid,),
      in_specs=[
          pl.BlockSpec((_BLK, _E), lambda i: (i, 0)),
          pl.BlockSpec((_BLK, _E), lambda i: (i, 0)),
          pl.BlockSpec((_E, _E), lambda i: (0, 0)),
          pl.BlockSpec((_BLK, 1), lambda i: (i, 0)),
          pl.BlockSpec((_BLK, 1), lambda i: (i, 0)),
          pl.BlockSpec((128, _E), lambda i: (0, 0)),
      ],
      out_specs=pl.BlockSpec((_BLK,), lambda i: (i,)),
      out_shape=jax.ShapeDtypeStruct((_B,), jnp.float32),
  )(h_s, h_d, Q, s_id.reshape(_B, 1), d_id.reshape(_B, 1), h_tail)


@jax.jit
def kernel(s_id, d_id, h_static, Q):
  s_id = s_id.astype(jnp.int32)
  d_id = d_id.astype(jnp.int32)
  tab_t = h_static.T
  rows_s, rows_d, bpos_s, bpos_d, wts, wtd = _sc_gather(s_id, d_id, tab_t)
  hs_pad, hd_pad = _sc_scatter(rows_s, rows_d, bpos_s, bpos_d, wts, wtd)
  h_tail = jnp.zeros((128, _E), jnp.float32).at[: _N - _TAIL].set(
      h_static[_TAIL:])
  return _tc_compute(hs_pad[:_B], hd_pad[:_B], Q, s_id, d_id, h_tail)


# speedup vs baseline: 3.7249x; 1.0308x over previous
import functools

import jax
import jax.numpy as jnp
from jax import lax
from jax.experimental import pallas as pl
from jax.experimental.pallas import tpu as pltpu
from jax.experimental.pallas import tpu_sc as plsc

_N = 1000001
_E = 32
_B = 16384
_W = 32
_CH = 1024
_NCH = 976
_TAIL = 999936
_CAP = 784
_SLOT = 1536
_CCAP = 272
_PIECE = 2048
_DEAD = 1 << 20
_BLK = 2048


def _iota16():
  return lax.iota(jnp.int32, 16)


def _sc_gather_body(s_hbm, d_hbm, tab_hbm,
                    rows_s_out, rows_d_out, bpos_s_out, bpos_d_out,
                    wts_out, wtd_out,
                    chunk0, chunk1, piece_s, piece_d,
                    wl_idx_s, wl_bpos_s, wl_idx_d, wl_bpos_d,
                    cwl_loc_s, cwl_bpos_s, cwl_loc_d, cwl_bpos_d,
                    grp_s, grp_d,
                    sem_p, sem_c0, sem_c1, sem_f):
  w = lax.axis_index("s") * 2 + lax.axis_index("c")
  it16 = _iota16()

  sent_bp = _B + it16
  dead = jnp.full((16,), _DEAD, jnp.int32)
  for v in range(_CAP // 16):
    wl_bpos_s[pl.ds(v * 16, 16)] = sent_bp
    wl_idx_s[pl.ds(v * 16, 16)] = dead
    wl_idx_d[pl.ds(v * 16, 16)] = dead
  for v in range(_CCAP // 16):
    cwl_loc_s[pl.ds(v * 16, 16)] = jnp.zeros((16,), jnp.int32)
    cwl_loc_d[pl.ds(v * 16, 16)] = jnp.zeros((16,), jnp.int32)
  for h in range(2):
    pltpu.sync_copy(wl_bpos_s.at[pl.ds(0, _SLOT // 2)],
                    bpos_s_out.at[pl.ds(w * _SLOT + h * (_SLOT // 2),
                                        _SLOT // 2)])
    pltpu.sync_copy(wl_bpos_s.at[pl.ds(0, _SLOT // 2)],
                    bpos_d_out.at[pl.ds(w * _SLOT + h * (_SLOT // 2),
                                        _SLOT // 2)])
  for v in range(_CAP // 16):
    wl_bpos_d[pl.ds(v * 16, 16)] = sent_bp

  def fire(k, buf, sem):
    c = w + 32 * k
    pltpu.make_async_copy(
        tab_hbm.at[:, pl.ds(c * _CH, _CH)], buf, sem).start()

  fire(0, chunk0, sem_c0)
  fire(1, chunk1, sem_c1)

  zero_v = jnp.zeros((16,), jnp.int32)
  off_s = zero_v
  off_d = zero_v
  for p in range(_B // _PIECE):
    pltpu.sync_copy(s_hbm.at[pl.ds(p * _PIECE, _PIECE)], piece_s)
    pltpu.sync_copy(d_hbm.at[pl.ds(p * _PIECE, _PIECE)], piece_d)

    def scan_vec(v, carry):
      o_s, o_d = carry
      idx_s = piece_s[pl.ds(v * 16, 16)]
      idx_d = piece_d[pl.ds(v * 16, 16)]
      m_s = ((idx_s >> 10) & 31) == w
      m_d = ((idx_d >> 10) & 31) == w
      cs_s = plsc.cumsum(m_s.astype(jnp.int32))
      cs_d = plsc.cumsum(m_d.astype(jnp.int32))
      pos_s = o_s + cs_s - 1
      pos_d = o_d + cs_d - 1
      bp = jnp.full((16,), p * _PIECE, jnp.int32) + v * 16 + it16
      plsc.store_scatter(wl_idx_s, [pos_s], idx_s, mask=m_s)
      plsc.store_scatter(wl_bpos_s, [pos_s], bp, mask=m_s)
      plsc.store_scatter(wl_idx_d, [pos_d], idx_d, mask=m_d)
      plsc.store_scatter(wl_bpos_d, [pos_d], bp, mask=m_d)
      n_s = plsc.all_reduce_population_count(m_s)
      n_d = plsc.all_reduce_population_count(m_d)
      return o_s + n_s, o_d + n_d

    off_s, off_d = lax.fori_loop(0, _PIECE // 16, scan_vec, (off_s, off_d))
  n_s = off_s[0]
  n_d = off_d[0]

  def process(c, buf, tot_s, tot_d):
    for v in range(_CCAP // 16):
      cwl_bpos_s[pl.ds(v * 16, 16)] = sent_bp
      cwl_bpos_d[pl.ds(v * 16, 16)] = sent_bp

    def scan_wl(v, carry):
      o_s, o_d = carry
      wi_s = wl_idx_s[pl.ds(v * 16, 16)]
      wi_d = wl_idx_d[pl.ds(v * 16, 16)]
      m_s = (wi_s >> 10) == c
      m_d = (wi_d >> 10) == c
      cs_s = plsc.cumsum(m_s.astype(jnp.int32))
      cs_d = plsc.cumsum(m_d.astype(jnp.int32))
      pos_s = o_s + cs_s - 1
      pos_d = o_d + cs_d - 1
      plsc.store_scatter(cwl_loc_s, [pos_s], wi_s & 1023, mask=m_s)
      plsc.store_scatter(cwl_loc_d, [pos_d], wi_d & 1023, mask=m_d)
      bp_s = wl_bpos_s[pl.ds(v * 16, 16)]
      bp_d = wl_bpos_d[pl.ds(v * 16, 16)]
      plsc.store_scatter(cwl_bpos_s, [pos_s], bp_s, mask=m_s)
      plsc.store_scatter(cwl_bpos_d, [pos_d], bp_d, mask=m_d)
      return (o_s + plsc.all_reduce_population_count(m_s),
              o_d + plsc.all_reduce_population_count(m_d))

    n_vecs = (jnp.maximum(n_s, n_d) + 15) >> 4
    off2_s, off2_d = lax.fori_loop(0, n_vecs, scan_wl, (zero_v, zero_v))

    def make_group(grp, cwl_loc, cwl_bpos, rows_out, bpos_out, tot):
      def do_group(g, _):
        base = pl.multiple_of(w * _SLOT + tot + g * 16, 16)
        loc16 = cwl_loc[pl.ds(g * 16, 16)]
        for j in range(_E):
          vals = plsc.load_gather(
              buf, [jnp.full((16,), j, jnp.int32), loc16])
          plsc.store_scatter(grp, [g * (16 * _E) + it16 * _E + j], vals)
        dst = pl.multiple_of(base * _E, 16 * _E)
        pltpu.make_async_copy(
            grp.at[pl.ds(g * (16 * _E), 16 * _E)],
            rows_out.at[pl.ds(dst, 16 * _E)], sem_f).start()
        pltpu.make_async_copy(
            cwl_bpos.at[pl.ds(g * 16, 16)],
            bpos_out.at[pl.ds(base, 16)], sem_f).start()
        return ()

      def drain_group(g, _):
        base = pl.multiple_of(w * _SLOT + tot + g * 16, 16)
        dst = pl.multiple_of(base * _E, 16 * _E)
        pltpu.make_async_copy(
            grp.at[pl.ds(g * (16 * _E), 16 * _E)],
            rows_out.at[pl.ds(dst, 16 * _E)], sem_f).wait()
        pltpu.make_async_copy(
            cwl_bpos.at[pl.ds(g * 16, 16)],
            bpos_out.at[pl.ds(base, 16)], sem_f).wait()
        return ()

      return do_group, drain_group

    og_s = off2_s[0]
    og_d = off2_d[0]
    do_s, dr_s = make_group(grp_s, cwl_loc_s, cwl_bpos_s,
                            rows_s_out, bpos_s_out, tot_s)
    do_d, dr_d = make_group(grp_d, cwl_loc_d, cwl_bpos_d,
                            rows_d_out, bpos_d_out, tot_d)
    ngr_s = (og_s + 15) >> 4
    ngr_d = (og_d + 15) >> 4
    lax.fori_loop(0, ngr_s, do_s, ())
    lax.fori_loop(0, ngr_d, do_d, ())
    lax.fori_loop(0, ngr_s, dr_s, ())
    lax.fori_loop(0, ngr_d, dr_d, ())
    return tot_s + ((og_s + 15) & ~15), tot_d + ((og_d + 15) & ~15)

  bufs = (chunk0, chunk1)
  sems = (sem_c0, sem_c1)

  def chunk_pair(kk, carry):
    tot_s, tot_d = carry
    for par in range(2):
      k = 2 * kk + par
      b, sm = bufs[par], sems[par]
      pltpu.make_async_copy(tab_hbm.at[:, pl.ds(0, _CH)], b, sm).wait()
      c = w + 32 * k
      tot_s, tot_d = process(c, b, tot_s, tot_d)

      @pl.when(w + 32 * (k + 2) < _NCH)
      def _():
        fire(k + 2, b, sm)
    return tot_s, tot_d

  tot_s, tot_d = lax.fori_loop(0, 15, chunk_pair,
                               (jnp.int32(0), jnp.int32(0)))

  def write_tots(ts, td):
    grp_s[pl.ds(0, 16)] = jnp.full((16,), ts, jnp.int32).astype(jnp.float32)
    grp_s[pl.ds(16, 16)] = jnp.full((16,), td, jnp.int32).astype(jnp.float32)
    pltpu.sync_copy(grp_s.at[pl.ds(0, 8)], wts_out.at[pl.ds(w * 8, 8)])
    pltpu.sync_copy(grp_s.at[pl.ds(16, 8)], wtd_out.at[pl.ds(w * 8, 8)])

  @pl.when(w < 16)
  def _():
    b, sm = bufs[0], sems[0]
    pltpu.make_async_copy(tab_hbm.at[:, pl.ds(0, _CH)], b, sm).wait()
    c = w + 32 * 30
    ts, td = process(c, b, tot_s, tot_d)
    write_tots(ts, td)

  @pl.when(w == 16)
  def _():
    b, sm = bufs[0], sems[0]
    pltpu.make_async_copy(
        tab_hbm.at[:, pl.ds(976 * _CH, 512)],
        b.at[:, pl.ds(0, 512)], sm).start()
    pltpu.make_async_copy(
        tab_hbm.at[:, pl.ds(0, 512)], b.at[:, pl.ds(0, 512)], sm).wait()
    ts, td = process(jnp.int32(976), b, tot_s, tot_d)
    write_tots(ts, td)

  @pl.when(w > 16)
  def _():
    write_tots(tot_s, tot_d)


def _sc_gather(s_id, d_id, tab_t):
  mesh = plsc.VectorSubcoreMesh(core_axis_name="c", subcore_axis_name="s")
  rows_ty = jax.ShapeDtypeStruct((_W * _SLOT * _E,), jnp.float32)
  bpos_ty = jax.ShapeDtypeStruct((_W * _SLOT,), jnp.int32)
  cnt_ty = jax.ShapeDtypeStruct((_W * 8,), jnp.float32)
  fn = pl.kernel(
      _sc_gather_body,
      out_type=(rows_ty, rows_ty, bpos_ty, bpos_ty, cnt_ty, cnt_ty),
      mesh=mesh,
      compiler_params=pltpu.CompilerParams(needs_layout_passes=False),
      scratch_types=[
          pltpu.VMEM((_E, _CH), jnp.float32),
          pltpu.VMEM((_E, _CH), jnp.float32),
          pltpu.VMEM((_PIECE,), jnp.int32),
          pltpu.VMEM((_PIECE,), jnp.int32),
          pltpu.VMEM((_CAP,), jnp.int32),
          pltpu.VMEM((_CAP,), jnp.int32),
          pltpu.VMEM((_CAP,), jnp.int32),
          pltpu.VMEM((_CAP,), jnp.int32),
          pltpu.VMEM((_CCAP,), jnp.int32),
          pltpu.VMEM((_CCAP,), jnp.int32),
          pltpu.VMEM((_CCAP,), jnp.int32),
          pltpu.VMEM((_CCAP,), jnp.int32),
          pltpu.VMEM((_CCAP * _E,), jnp.float32),
          pltpu.VMEM((_CCAP * _E,), jnp.float32),
          pltpu.SemaphoreType.DMA,
          pltpu.SemaphoreType.DMA,
          pltpu.SemaphoreType.DMA,
          pltpu.SemaphoreType.DMA,
      ],
  )
  return fn(s_id, d_id, tab_t)


def _sc_scatter_body(rows_s, rows_d, bpos_s, bpos_d, wts, wtd,
                     hs_out, hd_out, rows_v, bpv, cnt_v, sem, sem2):
  w = lax.axis_index("s") * 2 + lax.axis_index("c")
  pltpu.sync_copy(wts.at[pl.ds(w * 8, 8)], cnt_v.at[pl.ds(0, 8)])
  pltpu.sync_copy(wtd.at[pl.ds(w * 8, 8)], cnt_v.at[pl.ds(8, 8)])
  cnts = cnt_v[pl.ds(0, 16)].astype(jnp.int32)
  streams = ((rows_s, bpos_s, hs_out, 0), (rows_d, bpos_d, hd_out, 1))
  ngs = [(cnts[8 * st] + 127) >> 7 for st in range(2)]

  def stage(st, rows, bpos):
    def go(j, _):
      pltpu.make_async_copy(
          rows.at[pl.ds(w * _SLOT + j * 128, 128)],
          rows_v.at[st * (_SLOT // 128) + j], sem2).start()
      pltpu.make_async_copy(
          bpos.at[pl.ds(w * _SLOT + j * 128, 128)],
          bpv.at[st * (_SLOT // 128) + j], sem2).start()
      return ()
    return go

  def fire(st, out, rows, bpos):
    def go(j, _):
      pltpu.make_async_copy(
          rows.at[pl.ds(w * _SLOT + j * 128, 128)],
          rows_v.at[st * (_SLOT // 128) + j], sem2).wait()
      pltpu.make_async_copy(
          bpos.at[pl.ds(w * _SLOT + j * 128, 128)],
          bpv.at[st * (_SLOT // 128) + j], sem2).wait()
      pltpu.make_async_copy(
          rows_v.at[st * (_SLOT // 128) + j],
          out.at[bpv.at[st * (_SLOT // 128) + j]], sem).start()
      return ()
    return go

  def drain(st, out):
    def go(j, _):
      pltpu.make_async_copy(
          rows_v.at[st * (_SLOT // 128) + j],
          out.at[bpv.at[st * (_SLOT // 128) + j]], sem).wait()
      return ()
    return go

  for st, (rows, bpos, out, _) in enumerate(streams):
    lax.fori_loop(0, ngs[st], stage(st, rows, bpos), ())
  for st, (rows, bpos, out, _) in enumerate(streams):
    lax.fori_loop(0, ngs[st], fire(st, out, rows, bpos), ())
  for st, (rows, bpos, out, _) in enumerate(streams):
    lax.fori_loop(0, ngs[st], drain(st, out), ())


def _sc_scatter(rows_s, rows_d, bpos_s, bpos_d, wts, wtd):
  mesh = plsc.VectorSubcoreMesh(core_axis_name="c", subcore_axis_name="s")
  out_ty = jax.ShapeDtypeStruct((_B + 16, _E), jnp.float32)
  fn = pl.kernel(
      _sc_scatter_body,
      out_type=(out_ty, out_ty),
      mesh=mesh,
      compiler_params=pltpu.CompilerParams(use_tc_tiling_on_sc=False,
                                           needs_layout_passes=False),
      scratch_types=[
          pltpu.VMEM((2 * (_SLOT // 128), 128, _E), jnp.float32),
          pltpu.VMEM((2 * (_SLOT // 128), 128), jnp.int32),
          pltpu.VMEM((16,), jnp.float32),
          pltpu.SemaphoreType.DMA,
          pltpu.SemaphoreType.DMA,
      ],
  )
  return fn(rows_s.reshape(_W * _SLOT, _E), rows_d.reshape(_W * _SLOT, _E),
            bpos_s, bpos_d, wts, wtd)


def _tc_compute_body(hs_ref, hd_ref, q_ref, sid_ref, did_ref, tail_ref,
                     out_ref):
  it = lax.broadcasted_iota(jnp.int32, (_BLK, 128), 1)

  def patch(h, ids):
    os = ids[...] - _TAIL
    oh = (os == it).astype(jnp.float32)
    fix = jnp.dot(oh, tail_ref[...], preferred_element_type=jnp.float32)
    return jnp.where(os >= 0, fix, h[...])

  hs = patch(hs_ref, sid_ref)
  hd = patch(hd_ref, did_ref)
  t = jnp.dot(hs, q_ref[...], preferred_element_type=jnp.float32)
  out_ref[...] = jnp.exp(jnp.sum(t * hd, axis=1))


def _tc_compute(h_s, h_d, Q, s_id, d_id, h_tail):
  grid = _B // _BLK
  return pl.pallas_call(
      _tc_compute_body,
      grid=(grid,),
      in_specs=[
          pl.BlockSpec((_BLK, _E), lambda i: (i, 0)),
          pl.BlockSpec((_BLK, _E), lambda i: (i, 0)),
          pl.BlockSpec((_E, _E), lambda i: (0, 0)),
          pl.BlockSpec((_BLK, 1), lambda i: (i, 0)),
          pl.BlockSpec((_BLK, 1), lambda i: (i, 0)),
          pl.BlockSpec((128, _E), lambda i: (0, 0)),
      ],
      out_specs=pl.BlockSpec((_BLK,), lambda i: (i,)),
      out_shape=jax.ShapeDtypeStruct((_B,), jnp.float32),
  )(h_s, h_d, Q, s_id.reshape(_B, 1), d_id.reshape(_B, 1), h_tail)


@jax.jit
def kernel(s_id, d_id, h_static, Q):
  s_id = s_id.astype(jnp.int32)
  d_id = d_id.astype(jnp.int32)
  tab_t = h_static.T
  rows_s, rows_d, bpos_s, bpos_d, wts, wtd = _sc_gather(s_id, d_id, tab_t)
  hs_pad, hd_pad = _sc_scatter(rows_s, rows_d, bpos_s, bpos_d, wts, wtd)
  h_tail = jnp.zeros((128, _E), jnp.float32).at[: _N - _TAIL].set(
      h_static[_TAIL:])
  return _tc_compute(hs_pad[:_B], hd_pad[:_B], Q, s_id, d_id, h_tail)


# speedup vs baseline: 3.8996x; 1.0469x over previous
import functools

import jax
import jax.numpy as jnp
from jax import lax
from jax.experimental import pallas as pl
from jax.experimental.pallas import tpu as pltpu
from jax.experimental.pallas import tpu_sc as plsc

_N = 1000001
_E = 32
_B = 16384
_W = 32
_CH = 1024
_NCH = 976
_TAIL = 999936
_CAP = 784
_SLOT = 1536
_CCAP = 272
_PIECE = 2048
_DEAD = 1 << 20
_BLK = 2048


def _iota16():
  return lax.iota(jnp.int32, 16)


def _sc_gather_body(s_hbm, d_hbm, tab_hbm, tail_hbm,
                    rows_s_out, rows_d_out, bpos_s_out, bpos_d_out,
                    wts_out, wtd_out,
                    chunk0, chunk1, tail_v, piece_s, piece_d,
                    wl_idx_s, wl_bpos_s, wl_idx_d, wl_bpos_d,
                    cwl_loc_s, cwl_bpos_s, cwl_loc_d, cwl_bpos_d,
                    grp_s, grp_d,
                    sem_p, sem_c0, sem_c1, sem_f):
  w = lax.axis_index("s") * 2 + lax.axis_index("c")
  it16 = _iota16()

  sent_bp = _B + it16
  dead = jnp.full((16,), _DEAD, jnp.int32)
  for v in range(_CAP // 16):
    wl_bpos_s[pl.ds(v * 16, 16)] = sent_bp
    wl_idx_s[pl.ds(v * 16, 16)] = dead
    wl_idx_d[pl.ds(v * 16, 16)] = dead
  for v in range(_CCAP // 16):
    cwl_loc_s[pl.ds(v * 16, 16)] = jnp.zeros((16,), jnp.int32)
    cwl_loc_d[pl.ds(v * 16, 16)] = jnp.zeros((16,), jnp.int32)
  for h in range(2):
    pltpu.sync_copy(wl_bpos_s.at[pl.ds(0, _SLOT // 2)],
                    bpos_s_out.at[pl.ds(w * _SLOT + h * (_SLOT // 2),
                                        _SLOT // 2)])
    pltpu.sync_copy(wl_bpos_s.at[pl.ds(0, _SLOT // 2)],
                    bpos_d_out.at[pl.ds(w * _SLOT + h * (_SLOT // 2),
                                        _SLOT // 2)])
  for v in range(_CAP // 16):
    wl_bpos_d[pl.ds(v * 16, 16)] = sent_bp

  def fire(k, buf, sem):
    c = w + 32 * k
    pltpu.make_async_copy(
        tab_hbm.at[:, pl.ds(c * _CH, _CH)], buf, sem).start()

  fire(0, chunk0, sem_c0)
  fire(1, chunk1, sem_c1)

  zero_v = jnp.zeros((16,), jnp.int32)
  off_s = zero_v
  off_d = zero_v
  for p in range(_B // _PIECE):
    pltpu.sync_copy(s_hbm.at[pl.ds(p * _PIECE, _PIECE)], piece_s)
    pltpu.sync_copy(d_hbm.at[pl.ds(p * _PIECE, _PIECE)], piece_d)

    def scan_vec(v, carry):
      o_s, o_d = carry
      idx_s = piece_s[pl.ds(v * 16, 16)]
      idx_d = piece_d[pl.ds(v * 16, 16)]
      m_s = ((idx_s >> 10) & 31) == w
      m_d = ((idx_d >> 10) & 31) == w
      cs_s = plsc.cumsum(m_s.astype(jnp.int32))
      cs_d = plsc.cumsum(m_d.astype(jnp.int32))
      pos_s = o_s + cs_s - 1
      pos_d = o_d + cs_d - 1
      bp = jnp.full((16,), p * _PIECE, jnp.int32) + v * 16 + it16
      plsc.store_scatter(wl_idx_s, [pos_s], idx_s, mask=m_s)
      plsc.store_scatter(wl_bpos_s, [pos_s], bp, mask=m_s)
      plsc.store_scatter(wl_idx_d, [pos_d], idx_d, mask=m_d)
      plsc.store_scatter(wl_bpos_d, [pos_d], bp, mask=m_d)
      n_s = plsc.all_reduce_population_count(m_s)
      n_d = plsc.all_reduce_population_count(m_d)
      return o_s + n_s, o_d + n_d

    off_s, off_d = lax.fori_loop(0, _PIECE // 16, scan_vec, (off_s, off_d))
  n_s = off_s[0]
  n_d = off_d[0]

  def process(c, buf, tot_s, tot_d, mode=0):
    for v in range(_CCAP // 16):
      cwl_bpos_s[pl.ds(v * 16, 16)] = sent_bp
      cwl_bpos_d[pl.ds(v * 16, 16)] = sent_bp

    def scan_wl(v, carry):
      o_s, o_d = carry
      wi_s = wl_idx_s[pl.ds(v * 16, 16)]
      wi_d = wl_idx_d[pl.ds(v * 16, 16)]
      m_s = (wi_s >> 10) == c
      m_d = (wi_d >> 10) == c
      if mode == 1:
        m_s &= (wi_s & 1023) < 512
        m_d &= (wi_d & 1023) < 512
      elif mode == 2:
        m_s &= (wi_s & 1023) >= 512
        m_d &= (wi_d & 1023) >= 512
      cs_s = plsc.cumsum(m_s.astype(jnp.int32))
      cs_d = plsc.cumsum(m_d.astype(jnp.int32))
      pos_s = o_s + cs_s - 1
      pos_d = o_d + cs_d - 1
      plsc.store_scatter(cwl_loc_s, [pos_s], wi_s & 1023, mask=m_s)
      plsc.store_scatter(cwl_loc_d, [pos_d], wi_d & 1023, mask=m_d)
      bp_s = wl_bpos_s[pl.ds(v * 16, 16)]
      bp_d = wl_bpos_d[pl.ds(v * 16, 16)]
      plsc.store_scatter(cwl_bpos_s, [pos_s], bp_s, mask=m_s)
      plsc.store_scatter(cwl_bpos_d, [pos_d], bp_d, mask=m_d)
      return (o_s + plsc.all_reduce_population_count(m_s),
              o_d + plsc.all_reduce_population_count(m_d))

    n_vecs = (jnp.maximum(n_s, n_d) + 15) >> 4
    off2_s, off2_d = lax.fori_loop(0, n_vecs, scan_wl, (zero_v, zero_v))

    def make_group(grp, cwl_loc, cwl_bpos, rows_out, bpos_out, tot):
      def do_group(g, _):
        base = pl.multiple_of(w * _SLOT + tot + g * 16, 16)
        loc16 = cwl_loc[pl.ds(g * 16, 16)]
        for j in range(_E):
          if mode == 2:
            vals = plsc.load_gather(
                tail_v, [(loc16 - 512) * _E + j])
          else:
            vals = plsc.load_gather(
                buf, [jnp.full((16,), j, jnp.int32), loc16])
          plsc.store_scatter(grp, [g * (16 * _E) + it16 * _E + j], vals)
        dst = pl.multiple_of(base * _E, 16 * _E)
        pltpu.make_async_copy(
            grp.at[pl.ds(g * (16 * _E), 16 * _E)],
            rows_out.at[pl.ds(dst, 16 * _E)], sem_f).start()
        pltpu.make_async_copy(
            cwl_bpos.at[pl.ds(g * 16, 16)],
            bpos_out.at[pl.ds(base, 16)], sem_f).start()
        return ()

      def drain_group(g, _):
        base = pl.multiple_of(w * _SLOT + tot + g * 16, 16)
        dst = pl.multiple_of(base * _E, 16 * _E)
        pltpu.make_async_copy(
            grp.at[pl.ds(g * (16 * _E), 16 * _E)],
            rows_out.at[pl.ds(dst, 16 * _E)], sem_f).wait()
        pltpu.make_async_copy(
            cwl_bpos.at[pl.ds(g * 16, 16)],
            bpos_out.at[pl.ds(base, 16)], sem_f).wait()
        return ()

      return do_group, drain_group

    og_s = off2_s[0]
    og_d = off2_d[0]
    do_s, dr_s = make_group(grp_s, cwl_loc_s, cwl_bpos_s,
                            rows_s_out, bpos_s_out, tot_s)
    do_d, dr_d = make_group(grp_d, cwl_loc_d, cwl_bpos_d,
                            rows_d_out, bpos_d_out, tot_d)
    ngr_s = (og_s + 15) >> 4
    ngr_d = (og_d + 15) >> 4
    lax.fori_loop(0, ngr_s, do_s, ())
    lax.fori_loop(0, ngr_d, do_d, ())
    lax.fori_loop(0, ngr_s, dr_s, ())
    lax.fori_loop(0, ngr_d, dr_d, ())
    return tot_s + ((og_s + 15) & ~15), tot_d + ((og_d + 15) & ~15)

  bufs = (chunk0, chunk1)
  sems = (sem_c0, sem_c1)

  def chunk_pair(kk, carry):
    tot_s, tot_d = carry
    for par in range(2):
      k = 2 * kk + par
      b, sm = bufs[par], sems[par]
      pltpu.make_async_copy(tab_hbm.at[:, pl.ds(0, _CH)], b, sm).wait()
      c = w + 32 * k
      tot_s, tot_d = process(c, b, tot_s, tot_d)

      @pl.when(w + 32 * (k + 2) < _NCH)
      def _():
        fire(k + 2, b, sm)
    return tot_s, tot_d

  tot_s, tot_d = lax.fori_loop(0, 15, chunk_pair,
                               (jnp.int32(0), jnp.int32(0)))

  def write_tots(ts, td):
    grp_s[pl.ds(0, 16)] = jnp.full((16,), ts, jnp.int32).astype(jnp.float32)
    grp_s[pl.ds(16, 16)] = jnp.full((16,), td, jnp.int32).astype(jnp.float32)
    pltpu.sync_copy(grp_s.at[pl.ds(0, 8)], wts_out.at[pl.ds(w * 8, 8)])
    pltpu.sync_copy(grp_s.at[pl.ds(16, 8)], wtd_out.at[pl.ds(w * 8, 8)])

  @pl.when(w < 16)
  def _():
    b, sm = bufs[0], sems[0]
    pltpu.make_async_copy(tab_hbm.at[:, pl.ds(0, _CH)], b, sm).wait()
    c = w + 32 * 30
    ts, td = process(c, b, tot_s, tot_d)
    write_tots(ts, td)

  @pl.when(w == 16)
  def _():
    b, sm = bufs[0], sems[0]
    pltpu.make_async_copy(
        tab_hbm.at[:, pl.ds(976 * _CH, 512)],
        b.at[:, pl.ds(0, 512)], sm).start()
    pltpu.sync_copy(tail_hbm, tail_v)
    pltpu.make_async_copy(
        tab_hbm.at[:, pl.ds(0, 512)], b.at[:, pl.ds(0, 512)], sm).wait()
    ts, td = process(jnp.int32(976), b, tot_s, tot_d, mode=1)
    ts, td = process(jnp.int32(976), b, ts, td, mode=2)
    write_tots(ts, td)

  @pl.when(w > 16)
  def _():
    write_tots(tot_s, tot_d)


def _sc_gather(s_id, d_id, tab_t, tail_flat):
  mesh = plsc.VectorSubcoreMesh(core_axis_name="c", subcore_axis_name="s")
  rows_ty = jax.ShapeDtypeStruct((_W * _SLOT * _E,), jnp.float32)
  bpos_ty = jax.ShapeDtypeStruct((_W * _SLOT,), jnp.int32)
  cnt_ty = jax.ShapeDtypeStruct((_W * 8,), jnp.float32)
  fn = pl.kernel(
      _sc_gather_body,
      out_type=(rows_ty, rows_ty, bpos_ty, bpos_ty, cnt_ty, cnt_ty),
      mesh=mesh,
      compiler_params=pltpu.CompilerParams(needs_layout_passes=False),
      scratch_types=[
          pltpu.VMEM((_E, _CH), jnp.float32),
          pltpu.VMEM((_E, _CH), jnp.float32),
          pltpu.VMEM((128 * _E,), jnp.float32),
          pltpu.VMEM((_PIECE,), jnp.int32),
          pltpu.VMEM((_PIECE,), jnp.int32),
          pltpu.VMEM((_CAP,), jnp.int32),
          pltpu.VMEM((_CAP,), jnp.int32),
          pltpu.VMEM((_CAP,), jnp.int32),
          pltpu.VMEM((_CAP,), jnp.int32),
          pltpu.VMEM((_CCAP,), jnp.int32),
          pltpu.VMEM((_CCAP,), jnp.int32),
          pltpu.VMEM((_CCAP,), jnp.int32),
          pltpu.VMEM((_CCAP,), jnp.int32),
          pltpu.VMEM((_CCAP * _E,), jnp.float32),
          pltpu.VMEM((_CCAP * _E,), jnp.float32),
          pltpu.SemaphoreType.DMA,
          pltpu.SemaphoreType.DMA,
          pltpu.SemaphoreType.DMA,
          pltpu.SemaphoreType.DMA,
      ],
  )
  return fn(s_id, d_id, tab_t, tail_flat)


def _sc_scatter_body(rows_s, rows_d, bpos_s, bpos_d, wts, wtd,
                     hs_out, hd_out, rows_v, bpv, cnt_v, sem, sem2):
  w = lax.axis_index("s") * 2 + lax.axis_index("c")
  pltpu.sync_copy(wts.at[pl.ds(w * 8, 8)], cnt_v.at[pl.ds(0, 8)])
  pltpu.sync_copy(wtd.at[pl.ds(w * 8, 8)], cnt_v.at[pl.ds(8, 8)])
  cnts = cnt_v[pl.ds(0, 16)].astype(jnp.int32)
  streams = ((rows_s, bpos_s, hs_out, 0), (rows_d, bpos_d, hd_out, 1))
  ngs = [(cnts[8 * st] + 127) >> 7 for st in range(2)]

  def stage(st, rows, bpos):
    def go(j, _):
      pltpu.make_async_copy(
          rows.at[pl.ds(w * _SLOT + j * 128, 128)],
          rows_v.at[st * (_SLOT // 128) + j], sem2).start()
      pltpu.make_async_copy(
          bpos.at[pl.ds(w * _SLOT + j * 128, 128)],
          bpv.at[st * (_SLOT // 128) + j], sem2).start()
      return ()
    return go

  def fire(st, out, rows, bpos):
    def go(j, _):
      pltpu.make_async_copy(
          rows.at[pl.ds(w * _SLOT + j * 128, 128)],
          rows_v.at[st * (_SLOT // 128) + j], sem2).wait()
      pltpu.make_async_copy(
          bpos.at[pl.ds(w * _SLOT + j * 128, 128)],
          bpv.at[st * (_SLOT // 128) + j], sem2).wait()
      pltpu.make_async_copy(
          rows_v.at[st * (_SLOT // 128) + j],
          out.at[bpv.at[st * (_SLOT // 128) + j]], sem).start()
      return ()
    return go

  def drain(st, out):
    def go(j, _):
      pltpu.make_async_copy(
          rows_v.at[st * (_SLOT // 128) + j],
          out.at[bpv.at[st * (_SLOT // 128) + j]], sem).wait()
      return ()
    return go

  for st, (rows, bpos, out, _) in enumerate(streams):
    lax.fori_loop(0, ngs[st], stage(st, rows, bpos), ())
  for st, (rows, bpos, out, _) in enumerate(streams):
    lax.fori_loop(0, ngs[st], fire(st, out, rows, bpos), ())
  for st, (rows, bpos, out, _) in enumerate(streams):
    lax.fori_loop(0, ngs[st], drain(st, out), ())


def _sc_scatter(rows_s, rows_d, bpos_s, bpos_d, wts, wtd):
  mesh = plsc.VectorSubcoreMesh(core_axis_name="c", subcore_axis_name="s")
  out_ty = jax.ShapeDtypeStruct((_B + 16, _E), jnp.float32)
  fn = pl.kernel(
      _sc_scatter_body,
      out_type=(out_ty, out_ty),
      mesh=mesh,
      compiler_params=pltpu.CompilerParams(use_tc_tiling_on_sc=False,
                                           needs_layout_passes=False),
      scratch_types=[
          pltpu.VMEM((2 * (_SLOT // 128), 128, _E), jnp.float32),
          pltpu.VMEM((2 * (_SLOT // 128), 128), jnp.int32),
          pltpu.VMEM((16,), jnp.float32),
          pltpu.SemaphoreType.DMA,
          pltpu.SemaphoreType.DMA,
      ],
  )
  return fn(rows_s.reshape(_W * _SLOT, _E), rows_d.reshape(_W * _SLOT, _E),
            bpos_s, bpos_d, wts, wtd)


def _tc_compute_body(hs_ref, hd_ref, q_ref, out_ref):
  t = jnp.dot(hs_ref[...], q_ref[...], preferred_element_type=jnp.float32)
  out_ref[...] = jnp.exp(jnp.sum(t * hd_ref[...], axis=1))


def _tc_compute(h_s, h_d, Q):
  grid = _B // _BLK
  return pl.pallas_call(
      _tc_compute_body,
      grid=(grid,),
      in_specs=[
          pl.BlockSpec((_BLK, _E), lambda i: (i, 0)),
          pl.BlockSpec((_BLK, _E), lambda i: (i, 0)),
          pl.BlockSpec((_E, _E), lambda i: (0, 0)),
      ],
      out_specs=pl.BlockSpec((_BLK,), lambda i: (i,)),
      out_shape=jax.ShapeDtypeStruct((_B,), jnp.float32),
  )(h_s, h_d, Q)


@jax.jit
def kernel(s_id, d_id, h_static, Q):
  s_id = s_id.astype(jnp.int32)
  d_id = d_id.astype(jnp.int32)
  tab_t = h_static.T
  tail_flat = jnp.zeros((128, _E), jnp.float32).at[: _N - _TAIL].set(
      h_static[_TAIL:]).reshape(128 * _E)
  rows_s, rows_d, bpos_s, bpos_d, wts, wtd = _sc_gather(
      s_id, d_id, tab_t, tail_flat)
  hs_pad, hd_pad = _sc_scatter(rows_s, rows_d, bpos_s, bpos_d, wts, wtd)
  return _tc_compute(hs_pad[:_B], hd_pad[:_B], Q)


# speedup vs baseline: 4.1194x; 1.0564x over previous
import functools

import jax
import jax.numpy as jnp
from jax import lax
from jax.experimental import pallas as pl
from jax.experimental.pallas import tpu as pltpu
from jax.experimental.pallas import tpu_sc as plsc

_N = 1000001
_E = 32
_B = 16384
_W = 32
_CH = 1024
_NCH = 976
_TAIL = 999936
_CAP = 784
_SLOT = 1536
_CCAP = 272
_PIECE = 2048
_DEAD = 1 << 20
_BLK = 2048


def _iota16():
  return lax.iota(jnp.int32, 16)


def _sc_gather_body(s_hbm, d_hbm, tab_hbm, tail_hbm,
                    rows_s_out, rows_d_out, bpos_s_out, bpos_d_out,
                    wts_out, wtd_out,
                    chunk0, chunk1, tail_v, piece_s, piece_d,
                    wl_idx_s, wl_bpos_s, wl_idx_d, wl_bpos_d,
                    cwl_loc_s, cwl_bpos_s, cwl_loc_d, cwl_bpos_d,
                    grp_s, grp_d,
                    sem_p, sem_c0, sem_c1, sem_f):
  w = lax.axis_index("s") * 2 + lax.axis_index("c")
  it16 = _iota16()

  sent_bp = _B + it16
  dead = jnp.full((16,), _DEAD, jnp.int32)
  for v in range(_CAP // 16):
    wl_bpos_s[pl.ds(v * 16, 16)] = sent_bp
    wl_idx_s[pl.ds(v * 16, 16)] = dead
    wl_idx_d[pl.ds(v * 16, 16)] = dead
  for v in range(_CCAP // 16):
    cwl_loc_s[pl.ds(v * 16, 16)] = jnp.zeros((16,), jnp.int32)
    cwl_loc_d[pl.ds(v * 16, 16)] = jnp.zeros((16,), jnp.int32)
  for h in range(2):
    pltpu.sync_copy(wl_bpos_s.at[pl.ds(0, _SLOT // 2)],
                    bpos_s_out.at[pl.ds(w * _SLOT + h * (_SLOT // 2),
                                        _SLOT // 2)])
    pltpu.sync_copy(wl_bpos_s.at[pl.ds(0, _SLOT // 2)],
                    bpos_d_out.at[pl.ds(w * _SLOT + h * (_SLOT // 2),
                                        _SLOT // 2)])
  for v in range(_CAP // 16):
    wl_bpos_d[pl.ds(v * 16, 16)] = sent_bp

  def fire(k, buf, sem):
    c = w + 32 * k
    pltpu.make_async_copy(
        tab_hbm.at[:, pl.ds(c * _CH, _CH)], buf, sem).start()

  fire(0, chunk0, sem_c0)
  fire(1, chunk1, sem_c1)

  zero_v = jnp.zeros((16,), jnp.int32)
  off_s = zero_v
  off_d = zero_v
  for p in range(_B // _PIECE):
    pltpu.sync_copy(s_hbm.at[pl.ds(p * _PIECE, _PIECE)], piece_s)
    pltpu.sync_copy(d_hbm.at[pl.ds(p * _PIECE, _PIECE)], piece_d)

    def scan_vec(v, carry):
      o_s, o_d = carry
      idx_s = piece_s[pl.ds(v * 16, 16)]
      idx_d = piece_d[pl.ds(v * 16, 16)]
      m_s = ((idx_s >> 10) & 31) == w
      m_d = ((idx_d >> 10) & 31) == w
      cs_s = plsc.cumsum(m_s.astype(jnp.int32))
      cs_d = plsc.cumsum(m_d.astype(jnp.int32))
      pos_s = o_s + cs_s - 1
      pos_d = o_d + cs_d - 1
      bp = jnp.full((16,), p * _PIECE, jnp.int32) + v * 16 + it16
      plsc.store_scatter(wl_idx_s, [pos_s], idx_s, mask=m_s)
      plsc.store_scatter(wl_bpos_s, [pos_s], bp, mask=m_s)
      plsc.store_scatter(wl_idx_d, [pos_d], idx_d, mask=m_d)
      plsc.store_scatter(wl_bpos_d, [pos_d], bp, mask=m_d)
      n_s = plsc.all_reduce_population_count(m_s)
      n_d = plsc.all_reduce_population_count(m_d)
      return o_s + n_s, o_d + n_d

    off_s, off_d = lax.fori_loop(0, _PIECE // 16, scan_vec, (off_s, off_d))
  n_s = off_s[0]
  n_d = off_d[0]

  def process(c, buf, tot_s, tot_d, mode=0):
    for v in range(_CCAP // 16):
      cwl_bpos_s[pl.ds(v * 16, 16)] = sent_bp
      cwl_bpos_d[pl.ds(v * 16, 16)] = sent_bp

    def scan_wl(v, carry):
      o_s, o_d = carry
      wi_s = wl_idx_s[pl.ds(v * 16, 16)]
      wi_d = wl_idx_d[pl.ds(v * 16, 16)]
      m_s = (wi_s >> 10) == c
      m_d = (wi_d >> 10) == c
      if mode == 1:
        m_s &= (wi_s & 1023) < 512
        m_d &= (wi_d & 1023) < 512
      elif mode == 2:
        m_s &= (wi_s & 1023) >= 512
        m_d &= (wi_d & 1023) >= 512
      cs_s = plsc.cumsum(m_s.astype(jnp.int32))
      cs_d = plsc.cumsum(m_d.astype(jnp.int32))
      pos_s = o_s + cs_s - 1
      pos_d = o_d + cs_d - 1
      plsc.store_scatter(cwl_loc_s, [pos_s], wi_s & 1023, mask=m_s)
      plsc.store_scatter(cwl_loc_d, [pos_d], wi_d & 1023, mask=m_d)
      bp_s = wl_bpos_s[pl.ds(v * 16, 16)]
      bp_d = wl_bpos_d[pl.ds(v * 16, 16)]
      plsc.store_scatter(cwl_bpos_s, [pos_s], bp_s, mask=m_s)
      plsc.store_scatter(cwl_bpos_d, [pos_d], bp_d, mask=m_d)
      return (o_s + plsc.all_reduce_population_count(m_s),
              o_d + plsc.all_reduce_population_count(m_d))

    n_vecs = (jnp.maximum(n_s, n_d) + 15) >> 4
    off2_s, off2_d = lax.fori_loop(0, n_vecs, scan_wl, (zero_v, zero_v))

    def make_group(grp, cwl_loc, cwl_bpos, rows_out, bpos_out, tot):
      def do_group(g, _):
        base = pl.multiple_of(w * _SLOT + tot + g * 16, 16)
        loc16 = cwl_loc[pl.ds(g * 16, 16)]
        for j in range(_E):
          if mode == 2:
            vals = plsc.load_gather(
                tail_v, [(loc16 - 512) * _E + j])
          else:
            vals = plsc.load_gather(
                buf, [jnp.full((16,), j, jnp.int32), loc16])
          plsc.store_scatter(grp, [g * (16 * _E) + it16 * _E + j], vals)
        dst = pl.multiple_of(base * _E, 16 * _E)
        pltpu.make_async_copy(
            grp.at[pl.ds(g * (16 * _E), 16 * _E)],
            rows_out.at[pl.ds(dst, 16 * _E)], sem_f).start()
        pltpu.make_async_copy(
            cwl_bpos.at[pl.ds(g * 16, 16)],
            bpos_out.at[pl.ds(base, 16)], sem_f).start()
        return ()

      def drain_group(g, _):
        base = pl.multiple_of(w * _SLOT + tot + g * 16, 16)
        dst = pl.multiple_of(base * _E, 16 * _E)
        pltpu.make_async_copy(
            grp.at[pl.ds(g * (16 * _E), 16 * _E)],
            rows_out.at[pl.ds(dst, 16 * _E)], sem_f).wait()
        pltpu.make_async_copy(
            cwl_bpos.at[pl.ds(g * 16, 16)],
            bpos_out.at[pl.ds(base, 16)], sem_f).wait()
        return ()

      return do_group, drain_group

    og_s = off2_s[0]
    og_d = off2_d[0]
    do_s, dr_s = make_group(grp_s, cwl_loc_s, cwl_bpos_s,
                            rows_s_out, bpos_s_out, tot_s)
    do_d, dr_d = make_group(grp_d, cwl_loc_d, cwl_bpos_d,
                            rows_d_out, bpos_d_out, tot_d)
    ngr_s = (og_s + 15) >> 4
    ngr_d = (og_d + 15) >> 4
    lax.fori_loop(0, ngr_s, do_s, ())
    lax.fori_loop(0, ngr_d, do_d, ())
    lax.fori_loop(0, ngr_s, dr_s, ())
    lax.fori_loop(0, ngr_d, dr_d, ())
    return tot_s + ((og_s + 15) & ~15), tot_d + ((og_d + 15) & ~15)

  bufs = (chunk0, chunk1)
  sems = (sem_c0, sem_c1)

  def chunk_pair(kk, carry):
    tot_s, tot_d = carry
    for par in range(2):
      k = 2 * kk + par
      b, sm = bufs[par], sems[par]
      pltpu.make_async_copy(tab_hbm.at[:, pl.ds(0, _CH)], b, sm).wait()
      c = w + 32 * k
      tot_s, tot_d = process(c, b, tot_s, tot_d)

      @pl.when(w + 32 * (k + 2) < _NCH)
      def _():
        fire(k + 2, b, sm)
    return tot_s, tot_d

  tot_s, tot_d = lax.fori_loop(0, 15, chunk_pair,
                               (jnp.int32(0), jnp.int32(0)))

  def write_tots(ts, td):
    grp_s[pl.ds(0, 16)] = jnp.full((16,), ts, jnp.int32).astype(jnp.float32)
    grp_s[pl.ds(16, 16)] = jnp.full((16,), td, jnp.int32).astype(jnp.float32)
    pltpu.sync_copy(grp_s.at[pl.ds(0, 8)], wts_out.at[pl.ds(w * 8, 8)])
    pltpu.sync_copy(grp_s.at[pl.ds(16, 8)], wtd_out.at[pl.ds(w * 8, 8)])

  @pl.when(w < 16)
  def _():
    b, sm = bufs[0], sems[0]
    pltpu.make_async_copy(tab_hbm.at[:, pl.ds(0, _CH)], b, sm).wait()
    c = w + 32 * 30
    ts, td = process(c, b, tot_s, tot_d)
    write_tots(ts, td)

  @pl.when(w == 16)
  def _():
    b, sm = bufs[0], sems[0]
    pltpu.make_async_copy(
        tab_hbm.at[:, pl.ds(976 * _CH, 512)],
        b.at[:, pl.ds(0, 512)], sm).start()
    pltpu.sync_copy(tail_hbm, tail_v)
    pltpu.make_async_copy(
        tab_hbm.at[:, pl.ds(0, 512)], b.at[:, pl.ds(0, 512)], sm).wait()
    ts, td = process(jnp.int32(976), b, tot_s, tot_d, mode=1)
    ts, td = process(jnp.int32(976), b, ts, td, mode=2)
    write_tots(ts, td)

  @pl.when(w > 16)
  def _():
    write_tots(tot_s, tot_d)


def _sc_gather(s_id, d_id, tab_t, tail_flat):
  mesh = plsc.VectorSubcoreMesh(core_axis_name="c", subcore_axis_name="s")
  rows_ty = jax.ShapeDtypeStruct((_W * _SLOT * _E,), jnp.float32)
  bpos_ty = jax.ShapeDtypeStruct((_W * _SLOT,), jnp.int32)
  cnt_ty = jax.ShapeDtypeStruct((_W * 8,), jnp.float32)
  fn = pl.kernel(
      _sc_gather_body,
      out_type=(rows_ty, rows_ty, bpos_ty, bpos_ty, cnt_ty, cnt_ty),
      mesh=mesh,
      compiler_params=pltpu.CompilerParams(needs_layout_passes=False),
      scratch_types=[
          pltpu.VMEM((_E, _CH), jnp.float32),
          pltpu.VMEM((_E, _CH), jnp.float32),
          pltpu.VMEM((128 * _E,), jnp.float32),
          pltpu.VMEM((_PIECE,), jnp.int32),
          pltpu.VMEM((_PIECE,), jnp.int32),
          pltpu.VMEM((_CAP,), jnp.int32),
          pltpu.VMEM((_CAP,), jnp.int32),
          pltpu.VMEM((_CAP,), jnp.int32),
          pltpu.VMEM((_CAP,), jnp.int32),
          pltpu.VMEM((_CCAP,), jnp.int32),
          pltpu.VMEM((_CCAP,), jnp.int32),
          pltpu.VMEM((_CCAP,), jnp.int32),
          pltpu.VMEM((_CCAP,), jnp.int32),
          pltpu.VMEM((_CCAP * _E,), jnp.float32),
          pltpu.VMEM((_CCAP * _E,), jnp.float32),
          pltpu.SemaphoreType.DMA,
          pltpu.SemaphoreType.DMA,
          pltpu.SemaphoreType.DMA,
          pltpu.SemaphoreType.DMA,
      ],
  )
  return fn(s_id, d_id, tab_t, tail_flat)


def _sc_scatter_body(rows_s, rows_d, bpos_s, bpos_d, wts, wtd,
                     hs_out, hd_out, rows_v, bpv, cnt_v, sem, sem2):
  w = lax.axis_index("s") * 2 + lax.axis_index("c")
  pltpu.sync_copy(wts.at[pl.ds(w * 8, 8)], cnt_v.at[pl.ds(0, 8)])
  pltpu.sync_copy(wtd.at[pl.ds(w * 8, 8)], cnt_v.at[pl.ds(8, 8)])
  cnts = cnt_v[pl.ds(0, 16)].astype(jnp.int32)
  streams = ((rows_s, bpos_s, hs_out, 0), (rows_d, bpos_d, hd_out, 1))
  ngs = [(cnts[8 * st] + 127) >> 7 for st in range(2)]

  def stage(st, rows, bpos):
    def go(j, _):
      pltpu.make_async_copy(
          rows.at[pl.ds(w * _SLOT + j * 128, 128)],
          rows_v.at[st * (_SLOT // 128) + j], sem2).start()
      pltpu.make_async_copy(
          bpos.at[pl.ds(w * _SLOT + j * 128, 128)],
          bpv.at[st * (_SLOT // 128) + j], sem2).start()
      return ()
    return go

  def fire(st, out, rows, bpos):
    def go(j, _):
      pltpu.make_async_copy(
          rows.at[pl.ds(w * _SLOT + j * 128, 128)],
          rows_v.at[st * (_SLOT // 128) + j], sem2).wait()
      pltpu.make_async_copy(
          bpos.at[pl.ds(w * _SLOT + j * 128, 128)],
          bpv.at[st * (_SLOT // 128) + j], sem2).wait()
      pltpu.make_async_copy(
          rows_v.at[st * (_SLOT // 128) + j],
          out.at[bpv.at[st * (_SLOT // 128) + j]], sem).start()
      return ()
    return go

  def drain(st, out):
    def go(j, _):
      pltpu.make_async_copy(
          rows_v.at[st * (_SLOT // 128) + j],
          out.at[bpv.at[st * (_SLOT // 128) + j]], sem).wait()
      return ()
    return go

  for st, (rows, bpos, out, _) in enumerate(streams):
    lax.fori_loop(0, ngs[st], stage(st, rows, bpos), ())
  for st, (rows, bpos, out, _) in enumerate(streams):
    lax.fori_loop(0, ngs[st], fire(st, out, rows, bpos), ())
  for st, (rows, bpos, out, _) in enumerate(streams):
    lax.fori_loop(0, ngs[st], drain(st, out), ())


def _sc_scatter(rows_s, rows_d, bpos_s, bpos_d, wts, wtd):
  mesh = plsc.VectorSubcoreMesh(core_axis_name="c", subcore_axis_name="s")
  out_ty = jax.ShapeDtypeStruct((_B + 16, _E), jnp.float32)
  fn = pl.kernel(
      _sc_scatter_body,
      out_type=(out_ty, out_ty),
      mesh=mesh,
      compiler_params=pltpu.CompilerParams(use_tc_tiling_on_sc=False,
                                           needs_layout_passes=False),
      scratch_types=[
          pltpu.VMEM((2 * (_SLOT // 128), 128, _E), jnp.float32),
          pltpu.VMEM((2 * (_SLOT // 128), 128), jnp.int32),
          pltpu.VMEM((16,), jnp.float32),
          pltpu.SemaphoreType.DMA,
          pltpu.SemaphoreType.DMA,
      ],
  )
  return fn(rows_s.reshape(_W * _SLOT, _E), rows_d.reshape(_W * _SLOT, _E),
            bpos_s, bpos_d, wts, wtd)


def _tc_compute_body(hs_ref, hd_ref, q_ref, out_ref):
  t = jnp.dot(hs_ref[...], q_ref[...], preferred_element_type=jnp.float32)
  out_ref[...] = jnp.exp(jnp.sum(t * hd_ref[...], axis=1))


def _tc_compute(h_s, h_d, Q):
  grid = _B // _BLK
  return pl.pallas_call(
      _tc_compute_body,
      grid=(grid,),
      in_specs=[
          pl.BlockSpec((_BLK, _E), lambda i: (i, 0)),
          pl.BlockSpec((_BLK, _E), lambda i: (i, 0)),
          pl.BlockSpec((_E, _E), lambda i: (0, 0)),
      ],
      out_specs=pl.BlockSpec((_BLK,), lambda i: (i,)),
      out_shape=jax.ShapeDtypeStruct((_B,), jnp.float32),
  )(h_s, h_d, Q)


@jax.jit
def kernel(s_id, d_id, h_static, Q):
  s_id = s_id.astype(jnp.int32)
  d_id = d_id.astype(jnp.int32)
  tab_t = h_static.T
  tail_flat = jnp.zeros((128, _E), jnp.float32).at[: _N - _TAIL].set(
      h_static[_TAIL:]).reshape(128 * _E)
  rows_s, rows_d, bpos_s, bpos_d, wts, wtd = _sc_gather(
      s_id, d_id, tab_t, tail_flat)
  hs_pad, hd_pad = _sc_scatter(rows_s, rows_d, bpos_s, bpos_d, wts, wtd)
  return _tc_compute(hs_pad, hd_pad, Q)
